# Initial kernel scaffold; baseline (speedup 1.0000x reference)
import jax
import jax.numpy as jnp
from jax.experimental import pallas as pl


def kernel(features, adjM, ADJ, feature_attr, W_trans, b_trans, W_topo, b_topo, W_meta, b_meta, W_sem, b_sem, q_sem, W_lin, b_lin, feat_similar_neighbors):
    raise NotImplementedError("write your pallas kernel here")


# speedup vs baseline: 1.2554x; 1.2554x over previous
import functools

import jax
import jax.numpy as jnp
from jax import lax
from jax.experimental import pallas as pl
from jax.experimental.pallas import tpu as pltpu
from jax.experimental.pallas import tpu_sc as plsc

N = 10000
INFEAT = 256
HID = 128
TOPO = 32
NUMCLASS = 64

NW = 32
NPW = 320
NPAD = NW * NPW
CH = 4

HI = jax.lax.Precision.HIGHEST
MED = jax.lax.Precision.DEFAULT

RB1 = 2000
RB = 200
RB4 = 2000
RB5 = 2000


def _pre_body(f_ref, fa_ref, wt_ref, bt_ref, wtopo_ref, wm0_ref, wm1_ref,
              xt_ref, x0_ref, x1_ref, aw_ref):
    f = jnp.dot(f_ref[...], wt_ref[...], precision=HI) + bt_ref[...]
    xt_ref[...] = jnp.dot(f, wtopo_ref[...], precision=HI)
    x0_ref[...] = jnp.dot(f, wm0_ref[...], precision=HI)
    x1_ref[...] = jnp.dot(f, wm1_ref[...], precision=HI)
    a = fa_ref[...]
    e = jnp.exp(a - jnp.max(a, axis=1, keepdims=True))
    aw_ref[...] = e / jnp.sum(e, axis=1, keepdims=True)


def _precompute(features, fa, W_trans, bt, W_topo, wm0, wm1):
    gi = N // RB1
    return pl.pallas_call(
        _pre_body,
        grid=(gi,),
        in_specs=[
            pl.BlockSpec((RB1, INFEAT), lambda i: (i, 0)),
            pl.BlockSpec((RB1, TOPO), lambda i: (i, 0)),
            pl.BlockSpec((INFEAT, HID), lambda i: (0, 0)),
            pl.BlockSpec((1, HID), lambda i: (0, 0)),
            pl.BlockSpec((HID, HID), lambda i: (0, 0)),
            pl.BlockSpec((HID, HID), lambda i: (0, 0)),
            pl.BlockSpec((HID, HID), lambda i: (0, 0)),
        ],
        out_specs=[
            pl.BlockSpec((RB1, HID), lambda i: (i, 0)),
            pl.BlockSpec((RB1, HID), lambda i: (i, 0)),
            pl.BlockSpec((RB1, HID), lambda i: (i, 0)),
            pl.BlockSpec((RB1, TOPO), lambda i: (i, 0)),
        ],
        out_shape=[
            jax.ShapeDtypeStruct((N, HID), jnp.float32),
            jax.ShapeDtypeStruct((N, HID), jnp.float32),
            jax.ShapeDtypeStruct((N, HID), jnp.float32),
            jax.ShapeDtypeStruct((N, TOPO), jnp.float32),
        ],
    )(features, fa, W_trans, bt, W_topo, wm0, wm1)


def _sc_gather(xt, idx_flat, w_pad):
    info = plsc.get_sparse_core_info()
    nc = info.num_cores
    mesh = plsc.VectorSubcoreMesh(core_axis_name="c", subcore_axis_name="s")

    @functools.partial(
        pl.kernel, mesh=mesh,
        out_type=jax.ShapeDtypeStruct((NPAD, HID), jnp.float32),
        scratch_types=[
            pltpu.VMEM((CH * TOPO,), jnp.int32),
            pltpu.VMEM((CH * TOPO, HID), jnp.float32),
            pltpu.VMEM((CH, TOPO), jnp.float32),
            pltpu.VMEM((CH, HID), jnp.float32),
            pltpu.SemaphoreType.DMA,
        ],
    )
    def k(xt_hbm, idx_hbm, w_hbm, out_hbm, idx_v, rows_v, w_v, out_v, sem):
        wid = lax.axis_index("s") * nc + lax.axis_index("c")
        wbase = wid * NPW

        def chunk(ci, carry):
            node0 = wbase + ci * CH
            pltpu.sync_copy(idx_hbm.at[pl.ds(node0 * TOPO, CH * TOPO)], idx_v)
            pltpu.sync_copy(w_hbm.at[pl.ds(node0, CH)], w_v)
            pltpu.async_copy(xt_hbm.at[idx_v], rows_v, sem).wait()
            for n in range(CH):
                accs = [jnp.zeros((16,), jnp.float32) for _ in range(8)]
                for g in range(TOPO // 16):
                    wv = w_v[n, pl.ds(g * 16, 16)]
                    for j in range(16):
                        wgt = wv[j]
                        r = n * TOPO + g * 16 + j
                        for kk in range(8):
                            accs[kk] = accs[kk] + wgt * rows_v[r, pl.ds(kk * 16, 16)]
                for kk in range(8):
                    out_v[n, pl.ds(kk * 16, 16)] = accs[kk]
            pltpu.sync_copy(out_v, out_hbm.at[pl.ds(node0, CH)])
            return carry

        lax.fori_loop(0, NPW // CH, chunk, 0)

    return k(xt, idx_flat, w_pad)


def _big_body(adj_ref, a0_ref, a1_ref, xt_ref, x0_ref, x1_ref, bm0_ref, bm1_ref,
              aadj_ref, fmeta_ref):
    aadj_ref[...] = jnp.dot(adj_ref[...], xt_ref[...], precision=MED)
    m0 = jnp.dot(a0_ref[0], x0_ref[...], precision=MED)
    m1 = jnp.dot(a1_ref[0], x1_ref[...], precision=MED)
    fmeta_ref[...] = 0.5 * (jnp.tanh(m0 + bm0_ref[...]) +
                            jnp.tanh(m1 + bm1_ref[...]))


def _big(adjM, ADJ, xt, x0, x1, bm0, bm1):
    return pl.pallas_call(
        _big_body,
        grid=(N // RB,),
        in_specs=[
            pl.BlockSpec((RB, N), lambda i: (i, 0)),
            pl.BlockSpec((1, RB, N), lambda i: (0, i, 0)),
            pl.BlockSpec((1, RB, N), lambda i: (1, i, 0)),
            pl.BlockSpec((N, HID), lambda i: (0, 0)),
            pl.BlockSpec((N, HID), lambda i: (0, 0)),
            pl.BlockSpec((N, HID), lambda i: (0, 0)),
            pl.BlockSpec((1, HID), lambda i: (0, 0)),
            pl.BlockSpec((1, HID), lambda i: (0, 0)),
        ],
        out_specs=[
            pl.BlockSpec((RB, HID), lambda i: (i, 0)),
            pl.BlockSpec((RB, HID), lambda i: (i, 0)),
        ],
        out_shape=[
            jax.ShapeDtypeStruct((N, HID), jnp.float32),
            jax.ShapeDtypeStruct((N, HID), jnp.float32),
        ],
        compiler_params=pltpu.CompilerParams(
            dimension_semantics=("parallel",),
            vmem_limit_bytes=120 * 1024 * 1024),
    )(adjM, ADJ, ADJ, xt, x0, x1, bm0, bm1)


def _topo_body(aadj_ref, fs2_ref, fmeta_ref, btopo_ref, wsem_ref, bsem_ref,
               qsem_ref, ftopo_ref, wsum_ref):
    i = pl.program_id(0)

    @pl.when(i == 0)
    def _():
        wsum_ref[...] = jnp.zeros_like(wsum_ref)

    ftopo = jnp.tanh(aadj_ref[...] + fs2_ref[...] + btopo_ref[...])
    ftopo_ref[...] = ftopo
    sm = jnp.sum(jnp.tanh(jnp.dot(fmeta_ref[...], wsem_ref[...], precision=HI)
                          + bsem_ref[...]) * qsem_ref[...])
    st = jnp.sum(jnp.tanh(jnp.dot(ftopo, wsem_ref[...], precision=HI)
                          + bsem_ref[...]) * qsem_ref[...])
    upd = jnp.concatenate(
        [jnp.full((1, HID), sm, jnp.float32),
         jnp.full((1, HID), st, jnp.float32),
         jnp.zeros((6, HID), jnp.float32)], axis=0)
    wsum_ref[...] += upd


def _topo(aadj, fs2, fmeta, btopo, wsem, bsem, qsem):
    return pl.pallas_call(
        _topo_body,
        grid=(N // RB4,),
        in_specs=[
            pl.BlockSpec((RB4, HID), lambda i: (i, 0)),
            pl.BlockSpec((RB4, HID), lambda i: (i, 0)),
            pl.BlockSpec((RB4, HID), lambda i: (i, 0)),
            pl.BlockSpec((1, HID), lambda i: (0, 0)),
            pl.BlockSpec((HID, HID), lambda i: (0, 0)),
            pl.BlockSpec((1, HID), lambda i: (0, 0)),
            pl.BlockSpec((1, HID), lambda i: (0, 0)),
        ],
        out_specs=[
            pl.BlockSpec((RB4, HID), lambda i: (i, 0)),
            pl.BlockSpec((8, HID), lambda i: (0, 0)),
        ],
        out_shape=[
            jax.ShapeDtypeStruct((N, HID), jnp.float32),
            jax.ShapeDtypeStruct((8, HID), jnp.float32),
        ],
        compiler_params=pltpu.CompilerParams(
            dimension_semantics=("arbitrary",)),
    )(aadj, fs2, fmeta, btopo, wsem, bsem, qsem)


def _out_body(fmeta_ref, ftopo_ref, wsum_ref, wlin_ref, blin_ref,
              logits_ref, fout_ref):
    wm = wsum_ref[0, 0] * (1.0 / N)
    wt = wsum_ref[1, 0] * (1.0 / N)
    m = jnp.maximum(wm, wt)
    e0 = jnp.exp(wm - m)
    e1 = jnp.exp(wt - m)
    b0 = e0 / (e0 + e1)
    b1 = e1 / (e0 + e1)
    fo = b0 * fmeta_ref[...] + b1 * ftopo_ref[...]
    fout_ref[...] = fo
    logits_ref[...] = jnp.dot(fo, wlin_ref[...], precision=HI) + blin_ref[...]


def _final(fmeta, ftopo, wsum, wlin, blin):
    return pl.pallas_call(
        _out_body,
        grid=(N // RB5,),
        in_specs=[
            pl.BlockSpec((RB5, HID), lambda i: (i, 0)),
            pl.BlockSpec((RB5, HID), lambda i: (i, 0)),
            pl.BlockSpec((8, HID), lambda i: (0, 0)),
            pl.BlockSpec((HID, NUMCLASS), lambda i: (0, 0)),
            pl.BlockSpec((1, NUMCLASS), lambda i: (0, 0)),
        ],
        out_specs=[
            pl.BlockSpec((RB5, NUMCLASS), lambda i: (i, 0)),
            pl.BlockSpec((RB5, HID), lambda i: (i, 0)),
        ],
        out_shape=[
            jax.ShapeDtypeStruct((N, NUMCLASS), jnp.float32),
            jax.ShapeDtypeStruct((N, HID), jnp.float32),
        ],
    )(fmeta, ftopo, wsum, wlin, blin)


def kernel(features, adjM, ADJ, feature_attr, W_trans, b_trans, W_topo, b_topo,
           W_meta, b_meta, W_sem, b_sem, q_sem, W_lin, b_lin,
           feat_similar_neighbors):
    bt = b_trans.reshape(1, HID)
    bm0 = b_meta[0].reshape(1, HID)
    bm1 = b_meta[1].reshape(1, HID)
    btopo = b_topo.reshape(1, HID)
    bsem = b_sem.reshape(1, HID)
    qsem = q_sem.reshape(1, HID)
    blin = b_lin.reshape(1, NUMCLASS)

    xt, x0, x1, aw = _precompute(features, feature_attr, W_trans, bt, W_topo,
                                 W_meta[0], W_meta[1])

    idx = feat_similar_neighbors.astype(jnp.int32).reshape(-1)
    idx_pad = jnp.pad(idx, (0, (NPAD - N) * TOPO))
    aw_pad = jnp.pad(aw, ((0, NPAD - N), (0, 0)))
    fs2 = _sc_gather(xt, idx_pad, aw_pad)[:N]

    aadj, fmeta = _big(adjM, ADJ, xt, x0, x1, bm0, bm1)
    ftopo, wsum = _topo(aadj, fs2, fmeta, btopo, W_sem, bsem, qsem)
    logits, fout = _final(fmeta, ftopo, wsum, W_lin, blin)
    return (logits, fout)


# speedup vs baseline: 1.3889x; 1.1063x over previous
import functools

import jax
import jax.numpy as jnp
from jax import lax
from jax.experimental import pallas as pl
from jax.experimental.pallas import tpu as pltpu
from jax.experimental.pallas import tpu_sc as plsc

N = 10000
INFEAT = 256
HID = 128
TOPO = 32
NUMCLASS = 64

NW = 32
NPW = 320
NPAD = NW * NPW
CH = 4

RB1 = 2000
RB = 200
RB4 = 2000
RB5 = 2000


def _pre_body(f_ref, fa_ref, wt_ref, bt_ref, feat_ref, aw_ref):
    feat_ref[...] = f_ref[...] @ wt_ref[...] + bt_ref[...]
    a = fa_ref[...]
    e = jnp.exp(a - jnp.max(a, axis=1, keepdims=True))
    aw_ref[...] = e / jnp.sum(e, axis=1, keepdims=True)


def _precompute(features, fa, W_trans, bt):
    gi = N // RB1
    return pl.pallas_call(
        _pre_body,
        grid=(gi,),
        in_specs=[
            pl.BlockSpec((RB1, INFEAT), lambda i: (i, 0)),
            pl.BlockSpec((RB1, TOPO), lambda i: (i, 0)),
            pl.BlockSpec((INFEAT, HID), lambda i: (0, 0)),
            pl.BlockSpec((1, HID), lambda i: (0, 0)),
        ],
        out_specs=[
            pl.BlockSpec((RB1, HID), lambda i: (i, 0)),
            pl.BlockSpec((RB1, TOPO), lambda i: (i, 0)),
        ],
        out_shape=[
            jax.ShapeDtypeStruct((N, HID), jnp.float32),
            jax.ShapeDtypeStruct((N, TOPO), jnp.float32),
        ],
    )(features, fa, W_trans, bt)


NR = NPW * TOPO // 128


def _sc_gather(xt, idx_rows, w_pad):
    info = plsc.get_sparse_core_info()
    nc = info.num_cores
    mesh = plsc.VectorSubcoreMesh(core_axis_name="c", subcore_axis_name="s")

    @functools.partial(
        pl.kernel, mesh=mesh,
        out_type=jax.ShapeDtypeStruct((NPAD * HID,), jnp.float32),
        scratch_types=[
            pltpu.VMEM((128,), jnp.int32),
            pltpu.VMEM((128,), jnp.int32),
            pltpu.VMEM((NPW * TOPO,), jnp.float32),
            pltpu.VMEM((CH * TOPO, HID), jnp.float32),
            pltpu.VMEM((CH * TOPO, HID), jnp.float32),
            pltpu.VMEM((NPW * HID,), jnp.float32),
            pltpu.SemaphoreType.DMA,
            pltpu.SemaphoreType.DMA,
            pltpu.SemaphoreType.DMA,
            pltpu.SemaphoreType.DMA,
        ],
    )
    def k(xt_hbm, idx_hbm, w_hbm, out_hbm, idxb0, idxb1, w_v, rows0, rows1,
          out_v, si0, si1, sr0, sr1):
        wid = lax.axis_index("s") * nc + lax.axis_index("c")
        base = wid * NPW
        row0 = wid * NR
        pltpu.sync_copy(w_hbm.at[pl.ds(base * TOPO, NPW * TOPO)], w_v)
        bufs = ((idxb0, rows0, si0, sr0), (idxb1, rows1, si1, sr1))
        pltpu.sync_copy(idx_hbm.at[row0], idxb0)
        pltpu.make_async_copy(xt_hbm.at[idxb0], rows0, sr0).start()
        pltpu.sync_copy(idx_hbm.at[row0 + 1], idxb1)
        pltpu.make_async_copy(xt_hbm.at[idxb1], rows1, sr1).start()

        def pair(i, carry):
            c0 = i * 2
            for b in range(2):
                idxb, rows, si, sr = bufs[b]
                c = c0 + b
                pltpu.make_async_copy(xt_hbm.at[idxb], rows, sr).wait()

                @pl.when(c + 2 < NR)
                def _():
                    pltpu.make_async_copy(idx_hbm.at[row0 + c + 2], idxb, si).start()

                for n in range(CH):
                    node = c * CH + n
                    accs = [jnp.zeros((16,), jnp.float32) for _ in range(8)]
                    for g in range(TOPO // 16):
                        wv = w_v[pl.ds(node * TOPO + g * 16, 16)]
                        for j in range(16):
                            wgt = wv[j]
                            r = n * TOPO + g * 16 + j
                            for kk in range(8):
                                accs[kk] = accs[kk] + wgt * rows[r, pl.ds(kk * 16, 16)]
                    for kk in range(8):
                        out_v[pl.ds(node * HID + kk * 16, 16)] = accs[kk]

                @pl.when(c + 2 < NR)
                def _():
                    pltpu.make_async_copy(idx_hbm.at[row0 + c + 2], idxb, si).wait()
                    pltpu.make_async_copy(xt_hbm.at[idxb], rows, sr).start()
            return carry

        lax.fori_loop(0, NR // 2, pair, 0)
        pltpu.sync_copy(out_v, out_hbm.at[pl.ds(base * HID, NPW * HID)])

    return k(xt, idx_rows, w_pad)


def _big_body(adj_ref, a0_ref, a1_ref, feat_ref, wm0_ref, wm1_ref,
              bm0_ref, bm1_ref, aadj_ref, fmeta_ref):
    aadj_ref[...] = adj_ref[...] @ feat_ref[...]
    agg0 = a0_ref[0] @ feat_ref[...]
    agg1 = a1_ref[0] @ feat_ref[...]
    m0 = agg0 @ wm0_ref[...]
    m1 = agg1 @ wm1_ref[...]
    fmeta_ref[...] = 0.5 * (jnp.tanh(m0 + bm0_ref[...]) +
                            jnp.tanh(m1 + bm1_ref[...]))


def _big(adjM, ADJ, feat, wm0, wm1, bm0, bm1):
    return pl.pallas_call(
        _big_body,
        grid=(N // RB,),
        in_specs=[
            pl.BlockSpec((RB, N), lambda i: (i, 0)),
            pl.BlockSpec((1, RB, N), lambda i: (0, i, 0)),
            pl.BlockSpec((1, RB, N), lambda i: (1, i, 0)),
            pl.BlockSpec((N, HID), lambda i: (0, 0)),
            pl.BlockSpec((HID, HID), lambda i: (0, 0)),
            pl.BlockSpec((HID, HID), lambda i: (0, 0)),
            pl.BlockSpec((1, HID), lambda i: (0, 0)),
            pl.BlockSpec((1, HID), lambda i: (0, 0)),
        ],
        out_specs=[
            pl.BlockSpec((RB, HID), lambda i: (i, 0)),
            pl.BlockSpec((RB, HID), lambda i: (i, 0)),
        ],
        out_shape=[
            jax.ShapeDtypeStruct((N, HID), jnp.float32),
            jax.ShapeDtypeStruct((N, HID), jnp.float32),
        ],
        compiler_params=pltpu.CompilerParams(
            dimension_semantics=("parallel",),
            vmem_limit_bytes=120 * 1024 * 1024),
    )(adjM, ADJ, ADJ, feat, wm0, wm1, bm0, bm1)


def _topo_body(aadj_ref, fs_ref, fmeta_ref, wtopo_ref, btopo_ref, wsem_ref,
               bsem_ref, qsem_ref, ftopo_ref, wsum_ref):
    i = pl.program_id(0)

    @pl.when(i == 0)
    def _():
        wsum_ref[...] = jnp.zeros_like(wsum_ref)

    ftopo = jnp.tanh((aadj_ref[...] + fs_ref[...]) @ wtopo_ref[...]
                     + btopo_ref[...])
    ftopo_ref[...] = ftopo
    sm = jnp.sum(jnp.tanh(fmeta_ref[...] @ wsem_ref[...] + bsem_ref[...])
                 * qsem_ref[...])
    st = jnp.sum(jnp.tanh(ftopo @ wsem_ref[...] + bsem_ref[...])
                 * qsem_ref[...])
    upd = jnp.concatenate(
        [jnp.full((1, HID), sm, jnp.float32),
         jnp.full((1, HID), st, jnp.float32),
         jnp.zeros((6, HID), jnp.float32)], axis=0)
    wsum_ref[...] += upd


def _topo(aadj, fs, fmeta, wtopo, btopo, wsem, bsem, qsem):
    return pl.pallas_call(
        _topo_body,
        grid=(N // RB4,),
        in_specs=[
            pl.BlockSpec((RB4, HID), lambda i: (i, 0)),
            pl.BlockSpec((RB4, HID), lambda i: (i, 0)),
            pl.BlockSpec((RB4, HID), lambda i: (i, 0)),
            pl.BlockSpec((HID, HID), lambda i: (0, 0)),
            pl.BlockSpec((1, HID), lambda i: (0, 0)),
            pl.BlockSpec((HID, HID), lambda i: (0, 0)),
            pl.BlockSpec((1, HID), lambda i: (0, 0)),
            pl.BlockSpec((1, HID), lambda i: (0, 0)),
        ],
        out_specs=[
            pl.BlockSpec((RB4, HID), lambda i: (i, 0)),
            pl.BlockSpec((8, HID), lambda i: (0, 0)),
        ],
        out_shape=[
            jax.ShapeDtypeStruct((N, HID), jnp.float32),
            jax.ShapeDtypeStruct((8, HID), jnp.float32),
        ],
        compiler_params=pltpu.CompilerParams(
            dimension_semantics=("arbitrary",)),
    )(aadj, fs, fmeta, wtopo, btopo, wsem, bsem, qsem)


def _out_body(fmeta_ref, ftopo_ref, wsum_ref, wlin_ref, blin_ref,
              logits_ref, fout_ref):
    wm = wsum_ref[0, 0] * (1.0 / N)
    wt = wsum_ref[1, 0] * (1.0 / N)
    m = jnp.maximum(wm, wt)
    e0 = jnp.exp(wm - m)
    e1 = jnp.exp(wt - m)
    b0 = e0 / (e0 + e1)
    b1 = e1 / (e0 + e1)
    fo = b0 * fmeta_ref[...] + b1 * ftopo_ref[...]
    fout_ref[...] = fo
    logits_ref[...] = fo @ wlin_ref[...] + blin_ref[...]


def _final(fmeta, ftopo, wsum, wlin, blin):
    return pl.pallas_call(
        _out_body,
        grid=(N // RB5,),
        in_specs=[
            pl.BlockSpec((RB5, HID), lambda i: (i, 0)),
            pl.BlockSpec((RB5, HID), lambda i: (i, 0)),
            pl.BlockSpec((8, HID), lambda i: (0, 0)),
            pl.BlockSpec((HID, NUMCLASS), lambda i: (0, 0)),
            pl.BlockSpec((1, NUMCLASS), lambda i: (0, 0)),
        ],
        out_specs=[
            pl.BlockSpec((RB5, NUMCLASS), lambda i: (i, 0)),
            pl.BlockSpec((RB5, HID), lambda i: (i, 0)),
        ],
        out_shape=[
            jax.ShapeDtypeStruct((N, NUMCLASS), jnp.float32),
            jax.ShapeDtypeStruct((N, HID), jnp.float32),
        ],
    )(fmeta, ftopo, wsum, wlin, blin)


def kernel(features, adjM, ADJ, feature_attr, W_trans, b_trans, W_topo, b_topo,
           W_meta, b_meta, W_sem, b_sem, q_sem, W_lin, b_lin,
           feat_similar_neighbors):
    bt = b_trans.reshape(1, HID)
    bm0 = b_meta[0].reshape(1, HID)
    bm1 = b_meta[1].reshape(1, HID)
    btopo = b_topo.reshape(1, HID)
    bsem = b_sem.reshape(1, HID)
    qsem = q_sem.reshape(1, HID)
    blin = b_lin.reshape(1, NUMCLASS)

    feat, aw = _precompute(features, feature_attr, W_trans, bt)

    idx = feat_similar_neighbors.astype(jnp.int32).reshape(-1)
    idx_rows = jnp.pad(idx, (0, (NPAD - N) * TOPO)).reshape(NW * NR, 128)
    aw_pad = jnp.pad(aw, ((0, NPAD - N), (0, 0))).reshape(-1)
    fs = _sc_gather(feat, idx_rows, aw_pad).reshape(NPAD, HID)[:N]

    aadj, fmeta = _big(adjM, ADJ, feat, W_meta[0], W_meta[1], bm0, bm1)
    ftopo, wsum = _topo(aadj, fs, fmeta, W_topo, btopo, W_sem, bsem, qsem)
    logits, fout = _final(fmeta, ftopo, wsum, W_lin, blin)
    return (logits, fout)


# speedup vs baseline: 1.4128x; 1.0172x over previous
import functools

import jax
import jax.numpy as jnp
from jax import lax
from jax.experimental import pallas as pl
from jax.experimental.pallas import tpu as pltpu
from jax.experimental.pallas import tpu_sc as plsc

N = 10000
INFEAT = 256
HID = 128
TOPO = 32
NUMCLASS = 64

NW = 32
NPW = 320
NPAD = NW * NPW
CH = 4

RB1 = 2000
RB = 200
RB4 = 2000
RB5 = 2000


def _pre_body(f_ref, fa_ref, wt_ref, bt_ref, feat_ref, aw_ref):
    feat_ref[...] = f_ref[...] @ wt_ref[...] + bt_ref[...]
    a = fa_ref[...]
    e = jnp.exp(a - jnp.max(a, axis=1, keepdims=True))
    aw_ref[...] = e / jnp.sum(e, axis=1, keepdims=True)


def _precompute(features, fa, W_trans, bt):
    gi = N // RB1
    return pl.pallas_call(
        _pre_body,
        grid=(gi,),
        in_specs=[
            pl.BlockSpec((RB1, INFEAT), lambda i: (i, 0)),
            pl.BlockSpec((RB1, TOPO), lambda i: (i, 0)),
            pl.BlockSpec((INFEAT, HID), lambda i: (0, 0)),
            pl.BlockSpec((1, HID), lambda i: (0, 0)),
        ],
        out_specs=[
            pl.BlockSpec((RB1, HID), lambda i: (i, 0)),
            pl.BlockSpec((RB1, TOPO), lambda i: (i, 0)),
        ],
        out_shape=[
            jax.ShapeDtypeStruct((N, HID), jnp.float32),
            jax.ShapeDtypeStruct((N, TOPO), jnp.float32),
        ],
    )(features, fa, W_trans, bt)


NR = NPW * TOPO // 128


def _sc_gather(xt, idx_rows, w_pad):
    info = plsc.get_sparse_core_info()
    nc = info.num_cores
    mesh = plsc.VectorSubcoreMesh(core_axis_name="c", subcore_axis_name="s")

    NBUF = 4

    @functools.partial(
        pl.kernel, mesh=mesh,
        out_type=jax.ShapeDtypeStruct((NPAD * HID,), jnp.float32),
        scratch_types=(
            [pltpu.VMEM((128,), jnp.int32) for _ in range(NBUF)]
            + [pltpu.VMEM((CH * TOPO, HID), jnp.float32) for _ in range(NBUF)]
            + [pltpu.VMEM((NPW * TOPO,), jnp.float32),
               pltpu.VMEM((NPW * HID,), jnp.float32)]
            + [pltpu.SemaphoreType.DMA for _ in range(2 * NBUF)]
        ),
    )
    def k(xt_hbm, idx_hbm, w_hbm, out_hbm, *sc):
        idxbs = sc[0:NBUF]
        rowbs = sc[NBUF:2 * NBUF]
        w_v, out_v = sc[2 * NBUF], sc[2 * NBUF + 1]
        sis = sc[2 * NBUF + 2:2 * NBUF + 2 + NBUF]
        srs = sc[2 * NBUF + 2 + NBUF:2 * NBUF + 2 + 2 * NBUF]
        wid = lax.axis_index("s") * nc + lax.axis_index("c")
        base = wid * NPW
        row0 = wid * NR
        pltpu.sync_copy(w_hbm.at[pl.ds(base * TOPO, NPW * TOPO)], w_v)
        for b in range(NBUF):
            pltpu.sync_copy(idx_hbm.at[row0 + b], idxbs[b])
            pltpu.make_async_copy(xt_hbm.at[idxbs[b]], rowbs[b], srs[b]).start()

        def ring(i, carry):
            c0 = i * NBUF
            for b in range(NBUF):
                idxb, rows, si, sr = idxbs[b], rowbs[b], sis[b], srs[b]
                c = c0 + b
                pltpu.make_async_copy(xt_hbm.at[idxb], rows, sr).wait()

                @pl.when(c + NBUF < NR)
                def _():
                    pltpu.make_async_copy(idx_hbm.at[row0 + c + NBUF], idxb,
                                          si).start()

                def node_body(n, cr):
                    node = c * CH + n
                    accs = [jnp.zeros((16,), jnp.float32) for _ in range(8)]
                    for g in range(TOPO // 16):
                        wv = w_v[pl.ds(node * TOPO + g * 16, 16)]
                        for j in range(16):
                            wgt = wv[j]
                            r = n * TOPO + g * 16 + j
                            for kk in range(8):
                                accs[kk] = accs[kk] + wgt * rows[r, pl.ds(kk * 16, 16)]
                    for kk in range(8):
                        out_v[pl.ds(node * HID + kk * 16, 16)] = accs[kk]
                    return cr

                lax.fori_loop(0, CH, node_body, 0)

                @pl.when(c + NBUF < NR)
                def _():
                    pltpu.make_async_copy(idx_hbm.at[row0 + c + NBUF], idxb,
                                          si).wait()
                    pltpu.make_async_copy(xt_hbm.at[idxb], rows, sr).start()
            return carry

        lax.fori_loop(0, NR // NBUF, ring, 0)
        pltpu.sync_copy(out_v, out_hbm.at[pl.ds(base * HID, NPW * HID)])

    return k(xt, idx_rows, w_pad)


def _big_body(adj_ref, a0_ref, a1_ref, feat_ref, wm0_ref, wm1_ref,
              bm0_ref, bm1_ref, aadj_ref, fmeta_ref):
    aadj_ref[...] = adj_ref[...] @ feat_ref[...]
    agg0 = a0_ref[0] @ feat_ref[...]
    agg1 = a1_ref[0] @ feat_ref[...]
    m0 = agg0 @ wm0_ref[...]
    m1 = agg1 @ wm1_ref[...]
    fmeta_ref[...] = 0.5 * (jnp.tanh(m0 + bm0_ref[...]) +
                            jnp.tanh(m1 + bm1_ref[...]))


def _big(adjM, ADJ, feat, wm0, wm1, bm0, bm1):
    return pl.pallas_call(
        _big_body,
        grid=(N // RB,),
        in_specs=[
            pl.BlockSpec((RB, N), lambda i: (i, 0)),
            pl.BlockSpec((1, RB, N), lambda i: (0, i, 0)),
            pl.BlockSpec((1, RB, N), lambda i: (1, i, 0)),
            pl.BlockSpec((N, HID), lambda i: (0, 0)),
            pl.BlockSpec((HID, HID), lambda i: (0, 0)),
            pl.BlockSpec((HID, HID), lambda i: (0, 0)),
            pl.BlockSpec((1, HID), lambda i: (0, 0)),
            pl.BlockSpec((1, HID), lambda i: (0, 0)),
        ],
        out_specs=[
            pl.BlockSpec((RB, HID), lambda i: (i, 0)),
            pl.BlockSpec((RB, HID), lambda i: (i, 0)),
        ],
        out_shape=[
            jax.ShapeDtypeStruct((N, HID), jnp.float32),
            jax.ShapeDtypeStruct((N, HID), jnp.float32),
        ],
        compiler_params=pltpu.CompilerParams(
            dimension_semantics=("parallel",),
            vmem_limit_bytes=120 * 1024 * 1024),
    )(adjM, ADJ, ADJ, feat, wm0, wm1, bm0, bm1)


def _topo_body(aadj_ref, fs_ref, fmeta_ref, wtopo_ref, btopo_ref, wsem_ref,
               bsem_ref, qsem_ref, ftopo_ref, wsum_ref):
    i = pl.program_id(0)

    @pl.when(i == 0)
    def _():
        wsum_ref[...] = jnp.zeros_like(wsum_ref)

    ftopo = jnp.tanh((aadj_ref[...] + fs_ref[...]) @ wtopo_ref[...]
                     + btopo_ref[...])
    ftopo_ref[...] = ftopo
    sm = jnp.sum(jnp.tanh(fmeta_ref[...] @ wsem_ref[...] + bsem_ref[...])
                 * qsem_ref[...])
    st = jnp.sum(jnp.tanh(ftopo @ wsem_ref[...] + bsem_ref[...])
                 * qsem_ref[...])
    upd = jnp.concatenate(
        [jnp.full((1, HID), sm, jnp.float32),
         jnp.full((1, HID), st, jnp.float32),
         jnp.zeros((6, HID), jnp.float32)], axis=0)
    wsum_ref[...] += upd


def _topo(aadj, fs, fmeta, wtopo, btopo, wsem, bsem, qsem):
    return pl.pallas_call(
        _topo_body,
        grid=(N // RB4,),
        in_specs=[
            pl.BlockSpec((RB4, HID), lambda i: (i, 0)),
            pl.BlockSpec((RB4, HID), lambda i: (i, 0)),
            pl.BlockSpec((RB4, HID), lambda i: (i, 0)),
            pl.BlockSpec((HID, HID), lambda i: (0, 0)),
            pl.BlockSpec((1, HID), lambda i: (0, 0)),
            pl.BlockSpec((HID, HID), lambda i: (0, 0)),
            pl.BlockSpec((1, HID), lambda i: (0, 0)),
            pl.BlockSpec((1, HID), lambda i: (0, 0)),
        ],
        out_specs=[
            pl.BlockSpec((RB4, HID), lambda i: (i, 0)),
            pl.BlockSpec((8, HID), lambda i: (0, 0)),
        ],
        out_shape=[
            jax.ShapeDtypeStruct((N, HID), jnp.float32),
            jax.ShapeDtypeStruct((8, HID), jnp.float32),
        ],
        compiler_params=pltpu.CompilerParams(
            dimension_semantics=("arbitrary",)),
    )(aadj, fs, fmeta, wtopo, btopo, wsem, bsem, qsem)


def _out_body(fmeta_ref, ftopo_ref, wsum_ref, wlin_ref, blin_ref,
              logits_ref, fout_ref):
    wm = wsum_ref[0, 0] * (1.0 / N)
    wt = wsum_ref[1, 0] * (1.0 / N)
    m = jnp.maximum(wm, wt)
    e0 = jnp.exp(wm - m)
    e1 = jnp.exp(wt - m)
    b0 = e0 / (e0 + e1)
    b1 = e1 / (e0 + e1)
    fo = b0 * fmeta_ref[...] + b1 * ftopo_ref[...]
    fout_ref[...] = fo
    logits_ref[...] = fo @ wlin_ref[...] + blin_ref[...]


def _final(fmeta, ftopo, wsum, wlin, blin):
    return pl.pallas_call(
        _out_body,
        grid=(N // RB5,),
        in_specs=[
            pl.BlockSpec((RB5, HID), lambda i: (i, 0)),
            pl.BlockSpec((RB5, HID), lambda i: (i, 0)),
            pl.BlockSpec((8, HID), lambda i: (0, 0)),
            pl.BlockSpec((HID, NUMCLASS), lambda i: (0, 0)),
            pl.BlockSpec((1, NUMCLASS), lambda i: (0, 0)),
        ],
        out_specs=[
            pl.BlockSpec((RB5, NUMCLASS), lambda i: (i, 0)),
            pl.BlockSpec((RB5, HID), lambda i: (i, 0)),
        ],
        out_shape=[
            jax.ShapeDtypeStruct((N, NUMCLASS), jnp.float32),
            jax.ShapeDtypeStruct((N, HID), jnp.float32),
        ],
    )(fmeta, ftopo, wsum, wlin, blin)


def kernel(features, adjM, ADJ, feature_attr, W_trans, b_trans, W_topo, b_topo,
           W_meta, b_meta, W_sem, b_sem, q_sem, W_lin, b_lin,
           feat_similar_neighbors):
    bt = b_trans.reshape(1, HID)
    bm0 = b_meta[0].reshape(1, HID)
    bm1 = b_meta[1].reshape(1, HID)
    btopo = b_topo.reshape(1, HID)
    bsem = b_sem.reshape(1, HID)
    qsem = q_sem.reshape(1, HID)
    blin = b_lin.reshape(1, NUMCLASS)

    feat, aw = _precompute(features, feature_attr, W_trans, bt)

    idx = feat_similar_neighbors.astype(jnp.int32).reshape(-1)
    idx_rows = jnp.pad(idx, (0, (NPAD - N) * TOPO)).reshape(NW * NR, 128)
    aw_pad = jnp.pad(aw, ((0, NPAD - N), (0, 0))).reshape(-1)
    fs = _sc_gather(feat, idx_rows, aw_pad).reshape(NPAD, HID)[:N]

    aadj, fmeta = _big(adjM, ADJ, feat, W_meta[0], W_meta[1], bm0, bm1)
    ftopo, wsum = _topo(aadj, fs, fmeta, W_topo, btopo, W_sem, bsem, qsem)
    logits, fout = _final(fmeta, ftopo, wsum, W_lin, blin)
    return (logits, fout)


# speedup vs baseline: 1.7919x; 1.2683x over previous
import functools

import jax
import jax.numpy as jnp
from jax import lax
from jax.experimental import pallas as pl
from jax.experimental.pallas import tpu as pltpu
from jax.experimental.pallas import tpu_sc as plsc

N = 10000
INFEAT = 256
HID = 128
TOPO = 32
NUMCLASS = 64

NW = 32
NPW = 320
NPAD = NW * NPW
CH = 4

RB1 = 2000
RB = 200
RB4 = 2000
RB5 = 2000


def _pre_body(f_ref, fa_ref, wt_ref, bt_ref, feat_ref, aw_ref):
    feat_ref[...] = f_ref[...] @ wt_ref[...] + bt_ref[...]
    a = fa_ref[...]
    e = jnp.exp(a - jnp.max(a, axis=1, keepdims=True))
    aw_ref[...] = e / jnp.sum(e, axis=1, keepdims=True)


def _precompute(features, fa, W_trans, bt):
    gi = N // RB1
    return pl.pallas_call(
        _pre_body,
        grid=(gi,),
        in_specs=[
            pl.BlockSpec((RB1, INFEAT), lambda i: (i, 0)),
            pl.BlockSpec((RB1, TOPO), lambda i: (i, 0)),
            pl.BlockSpec((INFEAT, HID), lambda i: (0, 0)),
            pl.BlockSpec((1, HID), lambda i: (0, 0)),
        ],
        out_specs=[
            pl.BlockSpec((RB1, HID), lambda i: (i, 0)),
            pl.BlockSpec((RB1, TOPO), lambda i: (i, 0)),
        ],
        out_shape=[
            jax.ShapeDtypeStruct((N, HID), jnp.float32),
            jax.ShapeDtypeStruct((N, TOPO), jnp.float32),
        ],
    )(features, fa, W_trans, bt)


NR = NPW * TOPO // 128


def _sc_gather(xt, idx_rows, w_pad):
    info = plsc.get_sparse_core_info()
    nc = info.num_cores
    mesh = plsc.VectorSubcoreMesh(core_axis_name="c", subcore_axis_name="s")

    NBUF = 4

    @functools.partial(
        pl.kernel, mesh=mesh,
        compiler_params=pltpu.CompilerParams(use_tc_tiling_on_sc=False),
        out_type=jax.ShapeDtypeStruct((NPAD * HID,), jnp.float32),
        scratch_types=(
            [pltpu.VMEM((128,), jnp.int32) for _ in range(NBUF)]
            + [pltpu.VMEM((CH * TOPO, HID // 2), jnp.int32) for _ in range(NBUF)]
            + [pltpu.VMEM((NPW * TOPO,), jnp.float32),
               pltpu.VMEM((NPW * HID,), jnp.float32)]
            + [pltpu.SemaphoreType.DMA for _ in range(2 * NBUF)]
        ),
    )
    def k(xt_hbm, idx_hbm, w_hbm, out_hbm, *sc):
        idxbs = sc[0:NBUF]
        rowbs = sc[NBUF:2 * NBUF]
        w_v, out_v = sc[2 * NBUF], sc[2 * NBUF + 1]
        sis = sc[2 * NBUF + 2:2 * NBUF + 2 + NBUF]
        srs = sc[2 * NBUF + 2 + NBUF:2 * NBUF + 2 + 2 * NBUF]
        wid = lax.axis_index("s") * nc + lax.axis_index("c")
        base = wid * NPW
        row0 = wid * NR
        pltpu.sync_copy(w_hbm.at[pl.ds(base * TOPO, NPW * TOPO)], w_v)
        for b in range(NBUF):
            pltpu.sync_copy(idx_hbm.at[row0 + b], idxbs[b])
            pltpu.make_async_copy(xt_hbm.at[idxbs[b]], rowbs[b], srs[b]).start()

        def ring(i, carry):
            c0 = i * NBUF
            for b in range(NBUF):
                idxb, rows, si, sr = idxbs[b], rowbs[b], sis[b], srs[b]
                c = c0 + b
                pltpu.make_async_copy(xt_hbm.at[idxb], rows, sr).wait()

                @pl.when(c + NBUF < NR)
                def _():
                    pltpu.make_async_copy(idx_hbm.at[row0 + c + NBUF], idxb,
                                          si).start()

                def node_body(n, cr):
                    node = c * CH + n
                    accs = [jnp.zeros((16,), jnp.float32) for _ in range(8)]
                    for g in range(TOPO // 16):
                        wv = w_v[pl.ds(node * TOPO + g * 16, 16)]
                        for j in range(16):
                            wgt = wv[j]
                            r = n * TOPO + g * 16 + j
                            for g2 in range(HID // 32):
                                pv32 = rows[r, pl.ds(g2 * 16, 16)]
                                pa = jax.lax.bitcast_convert_type(
                                    pv32 << 16, jnp.float32)
                                pb = jax.lax.bitcast_convert_type(
                                    pv32 & jnp.int32(-65536), jnp.float32)
                                accs[2 * g2] = accs[2 * g2] + wgt * pa
                                accs[2 * g2 + 1] = accs[2 * g2 + 1] + wgt * pb
                    for kk in range(8):
                        out_v[pl.ds(node * HID + kk * 16, 16)] = accs[kk]
                    return cr

                lax.fori_loop(0, CH, node_body, 0)

                @pl.when(c + NBUF < NR)
                def _():
                    pltpu.make_async_copy(idx_hbm.at[row0 + c + NBUF], idxb,
                                          si).wait()
                    pltpu.make_async_copy(xt_hbm.at[idxb], rows, sr).start()
            return carry

        lax.fori_loop(0, NR // NBUF, ring, 0)
        pltpu.sync_copy(out_v, out_hbm.at[pl.ds(base * HID, NPW * HID)])

    return k(xt, idx_rows, w_pad)


def _big_body(adj_ref, a0_ref, a1_ref, feat_ref, wm0_ref, wm1_ref,
              bm0_ref, bm1_ref, aadj_ref, fmeta_ref):
    aadj_ref[...] = adj_ref[...] @ feat_ref[...]
    agg0 = a0_ref[0] @ feat_ref[...]
    agg1 = a1_ref[0] @ feat_ref[...]
    m0 = agg0 @ wm0_ref[...]
    m1 = agg1 @ wm1_ref[...]
    fmeta_ref[...] = 0.5 * (jnp.tanh(m0 + bm0_ref[...]) +
                            jnp.tanh(m1 + bm1_ref[...]))


def _big(adjM, ADJ, feat, wm0, wm1, bm0, bm1):
    return pl.pallas_call(
        _big_body,
        grid=(N // RB,),
        in_specs=[
            pl.BlockSpec((RB, N), lambda i: (i, 0)),
            pl.BlockSpec((1, RB, N), lambda i: (0, i, 0)),
            pl.BlockSpec((1, RB, N), lambda i: (1, i, 0)),
            pl.BlockSpec((N, HID), lambda i: (0, 0)),
            pl.BlockSpec((HID, HID), lambda i: (0, 0)),
            pl.BlockSpec((HID, HID), lambda i: (0, 0)),
            pl.BlockSpec((1, HID), lambda i: (0, 0)),
            pl.BlockSpec((1, HID), lambda i: (0, 0)),
        ],
        out_specs=[
            pl.BlockSpec((RB, HID), lambda i: (i, 0)),
            pl.BlockSpec((RB, HID), lambda i: (i, 0)),
        ],
        out_shape=[
            jax.ShapeDtypeStruct((N, HID), jnp.float32),
            jax.ShapeDtypeStruct((N, HID), jnp.float32),
        ],
        compiler_params=pltpu.CompilerParams(
            dimension_semantics=("parallel",),
            vmem_limit_bytes=120 * 1024 * 1024),
    )(adjM, ADJ, ADJ, feat, wm0, wm1, bm0, bm1)


def _topo_body(aadj_ref, fs_ref, fmeta_ref, wtopo_ref, btopo_ref, wsem_ref,
               bsem_ref, qsem_ref, ftopo_ref, wsum_ref):
    i = pl.program_id(0)

    @pl.when(i == 0)
    def _():
        wsum_ref[...] = jnp.zeros_like(wsum_ref)

    ftopo = jnp.tanh((aadj_ref[...] + fs_ref[...]) @ wtopo_ref[...]
                     + btopo_ref[...])
    ftopo_ref[...] = ftopo
    sm = jnp.sum(jnp.tanh(fmeta_ref[...] @ wsem_ref[...] + bsem_ref[...])
                 * qsem_ref[...])
    st = jnp.sum(jnp.tanh(ftopo @ wsem_ref[...] + bsem_ref[...])
                 * qsem_ref[...])
    upd = jnp.concatenate(
        [jnp.full((1, HID), sm, jnp.float32),
         jnp.full((1, HID), st, jnp.float32),
         jnp.zeros((6, HID), jnp.float32)], axis=0)
    wsum_ref[...] += upd


def _topo(aadj, fs, fmeta, wtopo, btopo, wsem, bsem, qsem):
    return pl.pallas_call(
        _topo_body,
        grid=(N // RB4,),
        in_specs=[
            pl.BlockSpec((RB4, HID), lambda i: (i, 0)),
            pl.BlockSpec((RB4, HID), lambda i: (i, 0)),
            pl.BlockSpec((RB4, HID), lambda i: (i, 0)),
            pl.BlockSpec((HID, HID), lambda i: (0, 0)),
            pl.BlockSpec((1, HID), lambda i: (0, 0)),
            pl.BlockSpec((HID, HID), lambda i: (0, 0)),
            pl.BlockSpec((1, HID), lambda i: (0, 0)),
            pl.BlockSpec((1, HID), lambda i: (0, 0)),
        ],
        out_specs=[
            pl.BlockSpec((RB4, HID), lambda i: (i, 0)),
            pl.BlockSpec((8, HID), lambda i: (0, 0)),
        ],
        out_shape=[
            jax.ShapeDtypeStruct((N, HID), jnp.float32),
            jax.ShapeDtypeStruct((8, HID), jnp.float32),
        ],
        compiler_params=pltpu.CompilerParams(
            dimension_semantics=("arbitrary",)),
    )(aadj, fs, fmeta, wtopo, btopo, wsem, bsem, qsem)


def _out_body(fmeta_ref, ftopo_ref, wsum_ref, wlin_ref, blin_ref,
              logits_ref, fout_ref):
    wm = wsum_ref[0, 0] * (1.0 / N)
    wt = wsum_ref[1, 0] * (1.0 / N)
    m = jnp.maximum(wm, wt)
    e0 = jnp.exp(wm - m)
    e1 = jnp.exp(wt - m)
    b0 = e0 / (e0 + e1)
    b1 = e1 / (e0 + e1)
    fo = b0 * fmeta_ref[...] + b1 * ftopo_ref[...]
    fout_ref[...] = fo
    logits_ref[...] = fo @ wlin_ref[...] + blin_ref[...]


def _final(fmeta, ftopo, wsum, wlin, blin):
    return pl.pallas_call(
        _out_body,
        grid=(N // RB5,),
        in_specs=[
            pl.BlockSpec((RB5, HID), lambda i: (i, 0)),
            pl.BlockSpec((RB5, HID), lambda i: (i, 0)),
            pl.BlockSpec((8, HID), lambda i: (0, 0)),
            pl.BlockSpec((HID, NUMCLASS), lambda i: (0, 0)),
            pl.BlockSpec((1, NUMCLASS), lambda i: (0, 0)),
        ],
        out_specs=[
            pl.BlockSpec((RB5, NUMCLASS), lambda i: (i, 0)),
            pl.BlockSpec((RB5, HID), lambda i: (i, 0)),
        ],
        out_shape=[
            jax.ShapeDtypeStruct((N, NUMCLASS), jnp.float32),
            jax.ShapeDtypeStruct((N, HID), jnp.float32),
        ],
    )(fmeta, ftopo, wsum, wlin, blin)


def kernel(features, adjM, ADJ, feature_attr, W_trans, b_trans, W_topo, b_topo,
           W_meta, b_meta, W_sem, b_sem, q_sem, W_lin, b_lin,
           feat_similar_neighbors):
    bt = b_trans.reshape(1, HID)
    bm0 = b_meta[0].reshape(1, HID)
    bm1 = b_meta[1].reshape(1, HID)
    btopo = b_topo.reshape(1, HID)
    bsem = b_sem.reshape(1, HID)
    qsem = q_sem.reshape(1, HID)
    blin = b_lin.reshape(1, NUMCLASS)

    feat, aw = _precompute(features, feature_attr, W_trans, bt)

    idx = feat_similar_neighbors.astype(jnp.int32).reshape(-1)
    idx_rows = jnp.pad(idx, (0, (NPAD - N) * TOPO)).reshape(NW * NR, 128)
    aw_pad = jnp.pad(aw, ((0, NPAD - N), (0, 0))).reshape(-1)
    feat_bf = (feat.reshape(N, HID // 32, 2, 16).transpose(0, 1, 3, 2)
               .reshape(N, HID // 2, 2).astype(jnp.bfloat16))
    feat_pk = jax.lax.bitcast_convert_type(feat_bf, jnp.int32)
    fs = _sc_gather(feat_pk, idx_rows, aw_pad).reshape(NPAD, HID)[:N]

    aadj, fmeta = _big(adjM, ADJ, feat, W_meta[0], W_meta[1], bm0, bm1)
    ftopo, wsum = _topo(aadj, fs, fmeta, W_topo, btopo, W_sem, bsem, qsem)
    logits, fout = _final(fmeta, ftopo, wsum, W_lin, blin)
    return (logits, fout)


# speedup vs baseline: 2.5141x; 1.4030x over previous
import functools

import jax
import jax.numpy as jnp
from jax import lax
from jax.experimental import pallas as pl
from jax.experimental.pallas import tpu as pltpu
from jax.experimental.pallas import tpu_sc as plsc

N = 10000
INFEAT = 256
HID = 128
TOPO = 32
NUMCLASS = 64

NW = 32
NPW = 320
NPAD = NW * NPW
CH = 4

RB1 = 2000
RB = 200
RB4 = 2000
RB5 = 2000


def _pre_body(f_ref, fa_ref, wt_ref, bt_ref, feat_ref, aw_ref):
    feat_ref[...] = f_ref[...] @ wt_ref[...] + bt_ref[...]
    a = fa_ref[...]
    e = jnp.exp(a - jnp.max(a, axis=1, keepdims=True))
    aw_ref[...] = e / jnp.sum(e, axis=1, keepdims=True)


def _precompute(features, fa, W_trans, bt):
    gi = N // RB1
    return pl.pallas_call(
        _pre_body,
        grid=(gi,),
        in_specs=[
            pl.BlockSpec((RB1, INFEAT), lambda i: (i, 0)),
            pl.BlockSpec((RB1, TOPO), lambda i: (i, 0)),
            pl.BlockSpec((INFEAT, HID), lambda i: (0, 0)),
            pl.BlockSpec((1, HID), lambda i: (0, 0)),
        ],
        out_specs=[
            pl.BlockSpec((RB1, HID), lambda i: (i, 0)),
            pl.BlockSpec((RB1, TOPO), lambda i: (i, 0)),
        ],
        out_shape=[
            jax.ShapeDtypeStruct((N, HID), jnp.float32),
            jax.ShapeDtypeStruct((N, TOPO), jnp.float32),
        ],
    )(features, fa, W_trans, bt)


NR = NPW * TOPO // 128


def _sc_gather(xt, idx_rows, w_pad):
    info = plsc.get_sparse_core_info()
    nc = info.num_cores
    mesh = plsc.VectorSubcoreMesh(core_axis_name="c", subcore_axis_name="s")

    NBUF = 4

    @functools.partial(
        pl.kernel, mesh=mesh,
        compiler_params=pltpu.CompilerParams(use_tc_tiling_on_sc=False),
        out_type=jax.ShapeDtypeStruct((NPAD * HID,), jnp.float32),
        scratch_types=(
            [pltpu.VMEM((128,), jnp.int32) for _ in range(NBUF)]
            + [pltpu.VMEM((CH * TOPO, HID // 2), jnp.int32) for _ in range(NBUF)]
            + [pltpu.VMEM((NPW * TOPO,), jnp.float32),
               pltpu.VMEM((NPW * HID,), jnp.float32),
               pltpu.VMEM_SHARED((N, HID // 2), jnp.int32)]
            + [pltpu.SemaphoreType.DMA for _ in range(2 * NBUF)]
        ),
    )
    def k(xt_hbm, idx_hbm, w_hbm, out_hbm, *sc):
        idxbs = sc[0:NBUF]
        rowbs = sc[NBUF:2 * NBUF]
        w_v, out_v, tbl = sc[2 * NBUF], sc[2 * NBUF + 1], sc[2 * NBUF + 2]
        sis = sc[2 * NBUF + 3:2 * NBUF + 3 + NBUF]
        srs = sc[2 * NBUF + 3 + NBUF:2 * NBUF + 3 + 2 * NBUF]
        wid = lax.axis_index("s") * nc + lax.axis_index("c")
        base = wid * NPW
        row0 = wid * NR

        @pl.when(lax.axis_index("s") == 0)
        def _():
            pltpu.sync_copy(xt_hbm, tbl)

        pltpu.sync_copy(w_hbm.at[pl.ds(base * TOPO, NPW * TOPO)], w_v)
        plsc.subcore_barrier()
        for b in range(NBUF):
            pltpu.sync_copy(idx_hbm.at[row0 + b], idxbs[b])
            pltpu.make_async_copy(tbl.at[idxbs[b]], rowbs[b], srs[b]).start()

        def ring(i, carry):
            c0 = i * NBUF
            for b in range(NBUF):
                idxb, rows, si, sr = idxbs[b], rowbs[b], sis[b], srs[b]
                c = c0 + b
                pltpu.make_async_copy(tbl.at[idxb], rows, sr).wait()

                @pl.when(c + NBUF < NR)
                def _():
                    pltpu.make_async_copy(idx_hbm.at[row0 + c + NBUF], idxb,
                                          si).start()

                def node_body(n, cr):
                    node = c * CH + n
                    accs = [jnp.zeros((16,), jnp.float32) for _ in range(8)]
                    for g in range(TOPO // 16):
                        wv = w_v[pl.ds(node * TOPO + g * 16, 16)]
                        for j in range(16):
                            wgt = wv[j]
                            r = n * TOPO + g * 16 + j
                            for g2 in range(HID // 32):
                                pv32 = rows[r, pl.ds(g2 * 16, 16)]
                                pa = jax.lax.bitcast_convert_type(
                                    pv32 << 16, jnp.float32)
                                pb = jax.lax.bitcast_convert_type(
                                    pv32 & jnp.int32(-65536), jnp.float32)
                                accs[2 * g2] = accs[2 * g2] + wgt * pa
                                accs[2 * g2 + 1] = accs[2 * g2 + 1] + wgt * pb
                    for kk in range(8):
                        out_v[pl.ds(node * HID + kk * 16, 16)] = accs[kk]
                    return cr

                lax.fori_loop(0, CH, node_body, 0)

                @pl.when(c + NBUF < NR)
                def _():
                    pltpu.make_async_copy(idx_hbm.at[row0 + c + NBUF], idxb,
                                          si).wait()
                    pltpu.make_async_copy(tbl.at[idxb], rows, sr).start()
            return carry

        lax.fori_loop(0, NR // NBUF, ring, 0)
        pltpu.sync_copy(out_v, out_hbm.at[pl.ds(base * HID, NPW * HID)])

    return k(xt, idx_rows, w_pad)


def _big_body(adj_ref, a0_ref, a1_ref, feat_ref, wm0_ref, wm1_ref,
              bm0_ref, bm1_ref, aadj_ref, fmeta_ref):
    aadj_ref[...] = adj_ref[...] @ feat_ref[...]
    agg0 = a0_ref[0] @ feat_ref[...]
    agg1 = a1_ref[0] @ feat_ref[...]
    m0 = agg0 @ wm0_ref[...]
    m1 = agg1 @ wm1_ref[...]
    fmeta_ref[...] = 0.5 * (jnp.tanh(m0 + bm0_ref[...]) +
                            jnp.tanh(m1 + bm1_ref[...]))


def _big(adjM, ADJ, feat, wm0, wm1, bm0, bm1):
    return pl.pallas_call(
        _big_body,
        grid=(N // RB,),
        in_specs=[
            pl.BlockSpec((RB, N), lambda i: (i, 0)),
            pl.BlockSpec((1, RB, N), lambda i: (0, i, 0)),
            pl.BlockSpec((1, RB, N), lambda i: (1, i, 0)),
            pl.BlockSpec((N, HID), lambda i: (0, 0)),
            pl.BlockSpec((HID, HID), lambda i: (0, 0)),
            pl.BlockSpec((HID, HID), lambda i: (0, 0)),
            pl.BlockSpec((1, HID), lambda i: (0, 0)),
            pl.BlockSpec((1, HID), lambda i: (0, 0)),
        ],
        out_specs=[
            pl.BlockSpec((RB, HID), lambda i: (i, 0)),
            pl.BlockSpec((RB, HID), lambda i: (i, 0)),
        ],
        out_shape=[
            jax.ShapeDtypeStruct((N, HID), jnp.float32),
            jax.ShapeDtypeStruct((N, HID), jnp.float32),
        ],
        compiler_params=pltpu.CompilerParams(
            dimension_semantics=("parallel",),
            vmem_limit_bytes=120 * 1024 * 1024),
    )(adjM, ADJ, ADJ, feat, wm0, wm1, bm0, bm1)


def _topo_body(aadj_ref, fs_ref, fmeta_ref, wtopo_ref, btopo_ref, wsem_ref,
               bsem_ref, qsem_ref, ftopo_ref, wsum_ref):
    i = pl.program_id(0)

    @pl.when(i == 0)
    def _():
        wsum_ref[...] = jnp.zeros_like(wsum_ref)

    ftopo = jnp.tanh((aadj_ref[...] + fs_ref[...]) @ wtopo_ref[...]
                     + btopo_ref[...])
    ftopo_ref[...] = ftopo
    sm = jnp.sum(jnp.tanh(fmeta_ref[...] @ wsem_ref[...] + bsem_ref[...])
                 * qsem_ref[...])
    st = jnp.sum(jnp.tanh(ftopo @ wsem_ref[...] + bsem_ref[...])
                 * qsem_ref[...])
    upd = jnp.concatenate(
        [jnp.full((1, HID), sm, jnp.float32),
         jnp.full((1, HID), st, jnp.float32),
         jnp.zeros((6, HID), jnp.float32)], axis=0)
    wsum_ref[...] += upd


def _topo(aadj, fs, fmeta, wtopo, btopo, wsem, bsem, qsem):
    return pl.pallas_call(
        _topo_body,
        grid=(N // RB4,),
        in_specs=[
            pl.BlockSpec((RB4, HID), lambda i: (i, 0)),
            pl.BlockSpec((RB4, HID), lambda i: (i, 0)),
            pl.BlockSpec((RB4, HID), lambda i: (i, 0)),
            pl.BlockSpec((HID, HID), lambda i: (0, 0)),
            pl.BlockSpec((1, HID), lambda i: (0, 0)),
            pl.BlockSpec((HID, HID), lambda i: (0, 0)),
            pl.BlockSpec((1, HID), lambda i: (0, 0)),
            pl.BlockSpec((1, HID), lambda i: (0, 0)),
        ],
        out_specs=[
            pl.BlockSpec((RB4, HID), lambda i: (i, 0)),
            pl.BlockSpec((8, HID), lambda i: (0, 0)),
        ],
        out_shape=[
            jax.ShapeDtypeStruct((N, HID), jnp.float32),
            jax.ShapeDtypeStruct((8, HID), jnp.float32),
        ],
        compiler_params=pltpu.CompilerParams(
            dimension_semantics=("arbitrary",)),
    )(aadj, fs, fmeta, wtopo, btopo, wsem, bsem, qsem)


def _out_body(fmeta_ref, ftopo_ref, wsum_ref, wlin_ref, blin_ref,
              logits_ref, fout_ref):
    wm = wsum_ref[0, 0] * (1.0 / N)
    wt = wsum_ref[1, 0] * (1.0 / N)
    m = jnp.maximum(wm, wt)
    e0 = jnp.exp(wm - m)
    e1 = jnp.exp(wt - m)
    b0 = e0 / (e0 + e1)
    b1 = e1 / (e0 + e1)
    fo = b0 * fmeta_ref[...] + b1 * ftopo_ref[...]
    fout_ref[...] = fo
    logits_ref[...] = fo @ wlin_ref[...] + blin_ref[...]


def _final(fmeta, ftopo, wsum, wlin, blin):
    return pl.pallas_call(
        _out_body,
        grid=(N // RB5,),
        in_specs=[
            pl.BlockSpec((RB5, HID), lambda i: (i, 0)),
            pl.BlockSpec((RB5, HID), lambda i: (i, 0)),
            pl.BlockSpec((8, HID), lambda i: (0, 0)),
            pl.BlockSpec((HID, NUMCLASS), lambda i: (0, 0)),
            pl.BlockSpec((1, NUMCLASS), lambda i: (0, 0)),
        ],
        out_specs=[
            pl.BlockSpec((RB5, NUMCLASS), lambda i: (i, 0)),
            pl.BlockSpec((RB5, HID), lambda i: (i, 0)),
        ],
        out_shape=[
            jax.ShapeDtypeStruct((N, NUMCLASS), jnp.float32),
            jax.ShapeDtypeStruct((N, HID), jnp.float32),
        ],
    )(fmeta, ftopo, wsum, wlin, blin)


def kernel(features, adjM, ADJ, feature_attr, W_trans, b_trans, W_topo, b_topo,
           W_meta, b_meta, W_sem, b_sem, q_sem, W_lin, b_lin,
           feat_similar_neighbors):
    bt = b_trans.reshape(1, HID)
    bm0 = b_meta[0].reshape(1, HID)
    bm1 = b_meta[1].reshape(1, HID)
    btopo = b_topo.reshape(1, HID)
    bsem = b_sem.reshape(1, HID)
    qsem = q_sem.reshape(1, HID)
    blin = b_lin.reshape(1, NUMCLASS)

    feat, aw = _precompute(features, feature_attr, W_trans, bt)

    idx = feat_similar_neighbors.astype(jnp.int32).reshape(-1)
    idx_rows = jnp.pad(idx, (0, (NPAD - N) * TOPO)).reshape(NW * NR, 128)
    aw_pad = jnp.pad(aw, ((0, NPAD - N), (0, 0))).reshape(-1)
    feat_bf = (feat.reshape(N, HID // 32, 2, 16).transpose(0, 1, 3, 2)
               .reshape(N, HID // 2, 2).astype(jnp.bfloat16))
    feat_pk = jax.lax.bitcast_convert_type(feat_bf, jnp.int32)
    fs = _sc_gather(feat_pk, idx_rows, aw_pad).reshape(NPAD, HID)[:N]

    aadj, fmeta = _big(adjM, ADJ, feat, W_meta[0], W_meta[1], bm0, bm1)
    ftopo, wsum = _topo(aadj, fs, fmeta, W_topo, btopo, W_sem, bsem, qsem)
    logits, fout = _final(fmeta, ftopo, wsum, W_lin, blin)
    return (logits, fout)


# speedup vs baseline: 2.5630x; 1.0194x over previous
import functools

import jax
import jax.numpy as jnp
from jax import lax
from jax.experimental import pallas as pl
from jax.experimental.pallas import tpu as pltpu
from jax.experimental.pallas import tpu_sc as plsc

N = 10000
INFEAT = 256
HID = 128
TOPO = 32
NUMCLASS = 64

NW = 32
NPW = 320
NPAD = NW * NPW
CH = 4

RB1 = 2000
RB = 200
RB4 = 2000
RB5 = 2000


def _pre_body(f_ref, fa_ref, wt_ref, bt_ref, feat_ref, pk_ref, aw_ref):
    f = f_ref[...] @ wt_ref[...] + bt_ref[...]
    feat_ref[...] = f
    u = jax.lax.bitcast_convert_type(f.astype(jnp.bfloat16),
                                     jnp.uint16).astype(jnp.int32)
    lo = jnp.concatenate([u[:, g * 32:g * 32 + 16] for g in range(HID // 32)],
                         axis=1)
    hi = jnp.concatenate([u[:, g * 32 + 16:g * 32 + 32]
                          for g in range(HID // 32)], axis=1)
    pk_ref[...] = lo | (hi << 16)
    a = fa_ref[...]
    e = jnp.exp(a - jnp.max(a, axis=1, keepdims=True))
    aw_ref[...] = e / jnp.sum(e, axis=1, keepdims=True)


def _precompute(features, fa, W_trans, bt):
    gi = N // RB1
    return pl.pallas_call(
        _pre_body,
        grid=(gi,),
        in_specs=[
            pl.BlockSpec((RB1, INFEAT), lambda i: (i, 0)),
            pl.BlockSpec((RB1, TOPO), lambda i: (i, 0)),
            pl.BlockSpec((INFEAT, HID), lambda i: (0, 0)),
            pl.BlockSpec((1, HID), lambda i: (0, 0)),
        ],
        out_specs=[
            pl.BlockSpec((RB1, HID), lambda i: (i, 0)),
            pl.BlockSpec((RB1, HID // 2), lambda i: (i, 0)),
            pl.BlockSpec((RB1, TOPO), lambda i: (i, 0)),
        ],
        out_shape=[
            jax.ShapeDtypeStruct((N, HID), jnp.float32),
            jax.ShapeDtypeStruct((N, HID // 2), jnp.int32),
            jax.ShapeDtypeStruct((N, TOPO), jnp.float32),
        ],
    )(features, fa, W_trans, bt)


NR = NPW * TOPO // 128


def _sc_gather(xt, idx_rows, w_pad):
    info = plsc.get_sparse_core_info()
    nc = info.num_cores
    mesh = plsc.VectorSubcoreMesh(core_axis_name="c", subcore_axis_name="s")

    NBUF = 4

    @functools.partial(
        pl.kernel, mesh=mesh,
        compiler_params=pltpu.CompilerParams(use_tc_tiling_on_sc=False),
        out_type=jax.ShapeDtypeStruct((NPAD * HID,), jnp.float32),
        scratch_types=(
            [pltpu.VMEM((128,), jnp.int32) for _ in range(NBUF)]
            + [pltpu.VMEM((CH * TOPO, HID // 2), jnp.int32) for _ in range(NBUF)]
            + [pltpu.VMEM((NPW * TOPO,), jnp.float32),
               pltpu.VMEM((NPW * HID,), jnp.float32),
               pltpu.VMEM_SHARED((N, HID // 2), jnp.int32)]
            + [pltpu.SemaphoreType.DMA for _ in range(2 * NBUF)]
        ),
    )
    def k(xt_hbm, idx_hbm, w_hbm, out_hbm, *sc):
        idxbs = sc[0:NBUF]
        rowbs = sc[NBUF:2 * NBUF]
        w_v, out_v, tbl = sc[2 * NBUF], sc[2 * NBUF + 1], sc[2 * NBUF + 2]
        sis = sc[2 * NBUF + 3:2 * NBUF + 3 + NBUF]
        srs = sc[2 * NBUF + 3 + NBUF:2 * NBUF + 3 + 2 * NBUF]
        wid = lax.axis_index("s") * nc + lax.axis_index("c")
        base = wid * NPW
        row0 = wid * NR

        @pl.when(lax.axis_index("s") == 0)
        def _():
            pltpu.sync_copy(xt_hbm, tbl)

        pltpu.sync_copy(w_hbm.at[pl.ds(base * TOPO, NPW * TOPO)], w_v)
        plsc.subcore_barrier()
        for b in range(NBUF):
            pltpu.sync_copy(idx_hbm.at[row0 + b], idxbs[b])
            pltpu.make_async_copy(tbl.at[idxbs[b]], rowbs[b], srs[b]).start()

        def ring(i, carry):
            c0 = i * NBUF
            for b in range(NBUF):
                idxb, rows, si, sr = idxbs[b], rowbs[b], sis[b], srs[b]
                c = c0 + b
                pltpu.make_async_copy(tbl.at[idxb], rows, sr).wait()

                @pl.when(c + NBUF < NR)
                def _():
                    pltpu.make_async_copy(idx_hbm.at[row0 + c + NBUF], idxb,
                                          si).start()

                def node_body(n, cr):
                    node = c * CH + n
                    accs = [jnp.zeros((16,), jnp.float32) for _ in range(8)]
                    for g in range(TOPO // 16):
                        wv = w_v[pl.ds(node * TOPO + g * 16, 16)]
                        for j in range(16):
                            wgt = wv[j]
                            r = n * TOPO + g * 16 + j
                            for g2 in range(HID // 32):
                                pv32 = rows[r, pl.ds(g2 * 16, 16)]
                                pa = jax.lax.bitcast_convert_type(
                                    pv32 << 16, jnp.float32)
                                pb = jax.lax.bitcast_convert_type(
                                    pv32 & jnp.int32(-65536), jnp.float32)
                                accs[2 * g2] = accs[2 * g2] + wgt * pa
                                accs[2 * g2 + 1] = accs[2 * g2 + 1] + wgt * pb
                    for kk in range(8):
                        out_v[pl.ds(node * HID + kk * 16, 16)] = accs[kk]
                    return cr

                lax.fori_loop(0, CH, node_body, 0)

                @pl.when(c + NBUF < NR)
                def _():
                    pltpu.make_async_copy(idx_hbm.at[row0 + c + NBUF], idxb,
                                          si).wait()
                    pltpu.make_async_copy(tbl.at[idxb], rows, sr).start()
            return carry

        lax.fori_loop(0, NR // NBUF, ring, 0)
        pltpu.sync_copy(out_v, out_hbm.at[pl.ds(base * HID, NPW * HID)])

    return k(xt, idx_rows, w_pad)


def _big_body(adj_ref, a0_ref, a1_ref, feat_ref, wm0_ref, wm1_ref,
              bm0_ref, bm1_ref, aadj_ref, fmeta_ref):
    aadj_ref[...] = adj_ref[...] @ feat_ref[...]
    agg0 = a0_ref[0] @ feat_ref[...]
    agg1 = a1_ref[0] @ feat_ref[...]
    m0 = agg0 @ wm0_ref[...]
    m1 = agg1 @ wm1_ref[...]
    fmeta_ref[...] = 0.5 * (jnp.tanh(m0 + bm0_ref[...]) +
                            jnp.tanh(m1 + bm1_ref[...]))


def _big(adjM, ADJ, feat, wm0, wm1, bm0, bm1):
    return pl.pallas_call(
        _big_body,
        grid=(N // RB,),
        in_specs=[
            pl.BlockSpec((RB, N), lambda i: (i, 0)),
            pl.BlockSpec((1, RB, N), lambda i: (0, i, 0)),
            pl.BlockSpec((1, RB, N), lambda i: (1, i, 0)),
            pl.BlockSpec((N, HID), lambda i: (0, 0)),
            pl.BlockSpec((HID, HID), lambda i: (0, 0)),
            pl.BlockSpec((HID, HID), lambda i: (0, 0)),
            pl.BlockSpec((1, HID), lambda i: (0, 0)),
            pl.BlockSpec((1, HID), lambda i: (0, 0)),
        ],
        out_specs=[
            pl.BlockSpec((RB, HID), lambda i: (i, 0)),
            pl.BlockSpec((RB, HID), lambda i: (i, 0)),
        ],
        out_shape=[
            jax.ShapeDtypeStruct((N, HID), jnp.float32),
            jax.ShapeDtypeStruct((N, HID), jnp.float32),
        ],
        compiler_params=pltpu.CompilerParams(
            dimension_semantics=("parallel",),
            vmem_limit_bytes=120 * 1024 * 1024),
    )(adjM, ADJ, ADJ, feat, wm0, wm1, bm0, bm1)


def _topo_body(aadj_ref, fs_ref, fmeta_ref, wtopo_ref, btopo_ref, wsem_ref,
               bsem_ref, qsem_ref, ftopo_ref, wsum_ref):
    i = pl.program_id(0)

    @pl.when(i == 0)
    def _():
        wsum_ref[...] = jnp.zeros_like(wsum_ref)

    ftopo = jnp.tanh((aadj_ref[...] + fs_ref[...]) @ wtopo_ref[...]
                     + btopo_ref[...])
    ftopo_ref[...] = ftopo
    sm = jnp.sum(jnp.tanh(fmeta_ref[...] @ wsem_ref[...] + bsem_ref[...])
                 * qsem_ref[...])
    st = jnp.sum(jnp.tanh(ftopo @ wsem_ref[...] + bsem_ref[...])
                 * qsem_ref[...])
    upd = jnp.concatenate(
        [jnp.full((1, HID), sm, jnp.float32),
         jnp.full((1, HID), st, jnp.float32),
         jnp.zeros((6, HID), jnp.float32)], axis=0)
    wsum_ref[...] += upd


def _topo(aadj, fs, fmeta, wtopo, btopo, wsem, bsem, qsem):
    return pl.pallas_call(
        _topo_body,
        grid=(N // RB4,),
        in_specs=[
            pl.BlockSpec((RB4, HID), lambda i: (i, 0)),
            pl.BlockSpec((RB4, HID), lambda i: (i, 0)),
            pl.BlockSpec((RB4, HID), lambda i: (i, 0)),
            pl.BlockSpec((HID, HID), lambda i: (0, 0)),
            pl.BlockSpec((1, HID), lambda i: (0, 0)),
            pl.BlockSpec((HID, HID), lambda i: (0, 0)),
            pl.BlockSpec((1, HID), lambda i: (0, 0)),
            pl.BlockSpec((1, HID), lambda i: (0, 0)),
        ],
        out_specs=[
            pl.BlockSpec((RB4, HID), lambda i: (i, 0)),
            pl.BlockSpec((8, HID), lambda i: (0, 0)),
        ],
        out_shape=[
            jax.ShapeDtypeStruct((N, HID), jnp.float32),
            jax.ShapeDtypeStruct((8, HID), jnp.float32),
        ],
        compiler_params=pltpu.CompilerParams(
            dimension_semantics=("arbitrary",)),
    )(aadj, fs, fmeta, wtopo, btopo, wsem, bsem, qsem)


def _out_body(fmeta_ref, ftopo_ref, wsum_ref, wlin_ref, blin_ref,
              logits_ref, fout_ref):
    wm = wsum_ref[0, 0] * (1.0 / N)
    wt = wsum_ref[1, 0] * (1.0 / N)
    m = jnp.maximum(wm, wt)
    e0 = jnp.exp(wm - m)
    e1 = jnp.exp(wt - m)
    b0 = e0 / (e0 + e1)
    b1 = e1 / (e0 + e1)
    fo = b0 * fmeta_ref[...] + b1 * ftopo_ref[...]
    fout_ref[...] = fo
    logits_ref[...] = fo @ wlin_ref[...] + blin_ref[...]


def _final(fmeta, ftopo, wsum, wlin, blin):
    return pl.pallas_call(
        _out_body,
        grid=(N // RB5,),
        in_specs=[
            pl.BlockSpec((RB5, HID), lambda i: (i, 0)),
            pl.BlockSpec((RB5, HID), lambda i: (i, 0)),
            pl.BlockSpec((8, HID), lambda i: (0, 0)),
            pl.BlockSpec((HID, NUMCLASS), lambda i: (0, 0)),
            pl.BlockSpec((1, NUMCLASS), lambda i: (0, 0)),
        ],
        out_specs=[
            pl.BlockSpec((RB5, NUMCLASS), lambda i: (i, 0)),
            pl.BlockSpec((RB5, HID), lambda i: (i, 0)),
        ],
        out_shape=[
            jax.ShapeDtypeStruct((N, NUMCLASS), jnp.float32),
            jax.ShapeDtypeStruct((N, HID), jnp.float32),
        ],
    )(fmeta, ftopo, wsum, wlin, blin)


def kernel(features, adjM, ADJ, feature_attr, W_trans, b_trans, W_topo, b_topo,
           W_meta, b_meta, W_sem, b_sem, q_sem, W_lin, b_lin,
           feat_similar_neighbors):
    bt = b_trans.reshape(1, HID)
    bm0 = b_meta[0].reshape(1, HID)
    bm1 = b_meta[1].reshape(1, HID)
    btopo = b_topo.reshape(1, HID)
    bsem = b_sem.reshape(1, HID)
    qsem = q_sem.reshape(1, HID)
    blin = b_lin.reshape(1, NUMCLASS)

    feat, feat_pk, aw = _precompute(features, feature_attr, W_trans, bt)

    idx = feat_similar_neighbors.astype(jnp.int32).reshape(-1)
    idx_rows = jnp.pad(idx, (0, (NPAD - N) * TOPO)).reshape(NW * NR, 128)
    aw_pad = jnp.pad(aw, ((0, NPAD - N), (0, 0))).reshape(-1)
    fs = _sc_gather(feat_pk, idx_rows, aw_pad).reshape(NPAD, HID)[:N]

    aadj, fmeta = _big(adjM, ADJ, feat, W_meta[0], W_meta[1], bm0, bm1)
    ftopo, wsum = _topo(aadj, fs, fmeta, W_topo, btopo, W_sem, bsem, qsem)
    logits, fout = _final(fmeta, ftopo, wsum, W_lin, blin)
    return (logits, fout)


# speedup vs baseline: 2.5824x; 1.0076x over previous
import functools

import jax
import jax.numpy as jnp
from jax import lax
from jax.experimental import pallas as pl
from jax.experimental.pallas import tpu as pltpu
from jax.experimental.pallas import tpu_sc as plsc

N = 10000
INFEAT = 256
HID = 128
TOPO = 32
NUMCLASS = 64

NW = 32
NPW = 320
NPAD = NW * NPW
CH = 4

RB1 = 2000
RB = 200
RB4 = 2000
RB5 = 2000


def _pre_body(f_ref, fa_ref, wt_ref, bt_ref, feat_ref, pk_ref, aw_ref):
    f = f_ref[...] @ wt_ref[...] + bt_ref[...]
    feat_ref[...] = f
    u = jax.lax.bitcast_convert_type(f.astype(jnp.bfloat16),
                                     jnp.uint16).astype(jnp.int32)
    lo = jnp.concatenate([u[:, g * 32:g * 32 + 16] for g in range(HID // 32)],
                         axis=1)
    hi = jnp.concatenate([u[:, g * 32 + 16:g * 32 + 32]
                          for g in range(HID // 32)], axis=1)
    pk_ref[...] = lo | (hi << 16)
    a = fa_ref[...]
    e = jnp.exp(a - jnp.max(a, axis=1, keepdims=True))
    aw_ref[...] = e / jnp.sum(e, axis=1, keepdims=True)


def _precompute(features, fa, W_trans, bt):
    gi = N // RB1
    return pl.pallas_call(
        _pre_body,
        grid=(gi,),
        in_specs=[
            pl.BlockSpec((RB1, INFEAT), lambda i: (i, 0)),
            pl.BlockSpec((RB1, TOPO), lambda i: (i, 0)),
            pl.BlockSpec((INFEAT, HID), lambda i: (0, 0)),
            pl.BlockSpec((1, HID), lambda i: (0, 0)),
        ],
        out_specs=[
            pl.BlockSpec((RB1, HID), lambda i: (i, 0)),
            pl.BlockSpec((RB1, HID // 2), lambda i: (i, 0)),
            pl.BlockSpec((RB1, TOPO), lambda i: (i, 0)),
        ],
        out_shape=[
            jax.ShapeDtypeStruct((N, HID), jnp.float32),
            jax.ShapeDtypeStruct((N, HID // 2), jnp.int32),
            jax.ShapeDtypeStruct((N, TOPO), jnp.float32),
        ],
    )(features, fa, W_trans, bt)


NR = NPW * TOPO // 128


def _sc_gather(xt, idx_rows, w_pad):
    info = plsc.get_sparse_core_info()
    nc = info.num_cores
    mesh = plsc.VectorSubcoreMesh(core_axis_name="c", subcore_axis_name="s")

    NBUF = 4

    @functools.partial(
        pl.kernel, mesh=mesh,
        compiler_params=pltpu.CompilerParams(use_tc_tiling_on_sc=False),
        out_type=jax.ShapeDtypeStruct((NPAD * HID,), jnp.float32),
        scratch_types=(
            [pltpu.VMEM((128,), jnp.int32) for _ in range(NBUF)]
            + [pltpu.VMEM((CH * TOPO, HID // 2), jnp.int32) for _ in range(NBUF)]
            + [pltpu.VMEM((NPW * TOPO,), jnp.float32),
               pltpu.VMEM((NPW * HID,), jnp.float32),
               pltpu.VMEM_SHARED((N, HID // 2), jnp.int32)]
            + [pltpu.SemaphoreType.DMA for _ in range(2 * NBUF)]
        ),
    )
    def k(xt_hbm, idx_hbm, w_hbm, out_hbm, *sc):
        idxbs = sc[0:NBUF]
        rowbs = sc[NBUF:2 * NBUF]
        w_v, out_v, tbl = sc[2 * NBUF], sc[2 * NBUF + 1], sc[2 * NBUF + 2]
        sis = sc[2 * NBUF + 3:2 * NBUF + 3 + NBUF]
        srs = sc[2 * NBUF + 3 + NBUF:2 * NBUF + 3 + 2 * NBUF]
        wid = lax.axis_index("s") * nc + lax.axis_index("c")
        base = wid * NPW
        row0 = wid * NR

        @pl.when(lax.axis_index("s") == 0)
        def _():
            pltpu.sync_copy(xt_hbm, tbl)

        pltpu.sync_copy(w_hbm.at[pl.ds(base * TOPO, NPW * TOPO)], w_v)
        plsc.subcore_barrier()
        for b in range(NBUF):
            pltpu.sync_copy(idx_hbm.at[row0 + b], idxbs[b])
            pltpu.make_async_copy(tbl.at[idxbs[b]], rowbs[b], srs[b]).start()

        def ring(i, carry):
            c0 = i * NBUF
            for b in range(NBUF):
                idxb, rows, si, sr = idxbs[b], rowbs[b], sis[b], srs[b]
                c = c0 + b
                pltpu.make_async_copy(tbl.at[idxb], rows, sr).wait()

                @pl.when(c + NBUF < NR)
                def _():
                    pltpu.make_async_copy(idx_hbm.at[row0 + c + NBUF], idxb,
                                          si).start()

                def node_body(n, cr):
                    node = c * CH + n
                    accs = [jnp.zeros((16,), jnp.float32) for _ in range(8)]
                    for g in range(TOPO // 16):
                        wv = w_v[pl.ds(node * TOPO + g * 16, 16)]
                        for j in range(16):
                            wgt = wv[j]
                            r = n * TOPO + g * 16 + j
                            for g2 in range(HID // 32):
                                pv32 = rows[r, pl.ds(g2 * 16, 16)]
                                pa = jax.lax.bitcast_convert_type(
                                    pv32 << 16, jnp.float32)
                                pb = jax.lax.bitcast_convert_type(
                                    pv32 & jnp.int32(-65536), jnp.float32)
                                accs[2 * g2] = accs[2 * g2] + wgt * pa
                                accs[2 * g2 + 1] = accs[2 * g2 + 1] + wgt * pb
                    for kk in range(8):
                        out_v[pl.ds(node * HID + kk * 16, 16)] = accs[kk]
                    return cr

                lax.fori_loop(0, CH, node_body, 0)

                @pl.when(c + NBUF < NR)
                def _():
                    pltpu.make_async_copy(idx_hbm.at[row0 + c + NBUF], idxb,
                                          si).wait()
                    pltpu.make_async_copy(tbl.at[idxb], rows, sr).start()
            return carry

        lax.fori_loop(0, NR // NBUF, ring, 0)
        pltpu.sync_copy(out_v, out_hbm.at[pl.ds(base * HID, NPW * HID)])

    return k(xt, idx_rows, w_pad)


def _big_body(adj_ref, a0_ref, a1_ref, feat_ref, wm0_ref, wm1_ref,
              bm0_ref, bm1_ref, aadj_ref, fmeta_ref):
    aadj_ref[...] = adj_ref[...] @ feat_ref[...]
    agg0 = a0_ref[0] @ feat_ref[...]
    agg1 = a1_ref[0] @ feat_ref[...]
    m0 = agg0 @ wm0_ref[...]
    m1 = agg1 @ wm1_ref[...]
    fmeta_ref[...] = 0.5 * (jnp.tanh(m0 + bm0_ref[...]) +
                            jnp.tanh(m1 + bm1_ref[...]))


def _big(adjM, ADJ, feat, wm0, wm1, bm0, bm1):
    return pl.pallas_call(
        _big_body,
        grid=(N // RB,),
        in_specs=[
            pl.BlockSpec((RB, N), lambda i: (i, 0)),
            pl.BlockSpec((1, RB, N), lambda i: (0, i, 0)),
            pl.BlockSpec((1, RB, N), lambda i: (1, i, 0)),
            pl.BlockSpec((N, HID), lambda i: (0, 0)),
            pl.BlockSpec((HID, HID), lambda i: (0, 0)),
            pl.BlockSpec((HID, HID), lambda i: (0, 0)),
            pl.BlockSpec((1, HID), lambda i: (0, 0)),
            pl.BlockSpec((1, HID), lambda i: (0, 0)),
        ],
        out_specs=[
            pl.BlockSpec((RB, HID), lambda i: (i, 0)),
            pl.BlockSpec((RB, HID), lambda i: (i, 0)),
        ],
        out_shape=[
            jax.ShapeDtypeStruct((N, HID), jnp.float32),
            jax.ShapeDtypeStruct((N, HID), jnp.float32),
        ],
        compiler_params=pltpu.CompilerParams(
            dimension_semantics=("parallel",),
            vmem_limit_bytes=120 * 1024 * 1024),
    )(adjM, ADJ, ADJ, feat, wm0, wm1, bm0, bm1)


def _topo_body(aadj_ref, fs_ref, fmeta_ref, wtopo_ref, btopo_ref, wsem_ref,
               bsem_ref, qsem_ref, ftopo_ref, wsum_ref):
    i = pl.program_id(0)

    @pl.when(i == 0)
    def _():
        wsum_ref[...] = jnp.zeros_like(wsum_ref)

    ftopo = jnp.tanh((aadj_ref[...] + fs_ref[...]) @ wtopo_ref[...]
                     + btopo_ref[...])
    ftopo_ref[...] = ftopo
    sm = jnp.sum(jnp.tanh(fmeta_ref[...] @ wsem_ref[...] + bsem_ref[...])
                 * qsem_ref[...])
    st = jnp.sum(jnp.tanh(ftopo @ wsem_ref[...] + bsem_ref[...])
                 * qsem_ref[...])
    upd = jnp.concatenate(
        [jnp.full((1, HID), sm, jnp.float32),
         jnp.full((1, HID), st, jnp.float32),
         jnp.zeros((6, HID), jnp.float32)], axis=0)
    wsum_ref[...] += upd


def _topo(aadj, fs, fmeta, wtopo, btopo, wsem, bsem, qsem):
    return pl.pallas_call(
        _topo_body,
        grid=(N // RB4,),
        in_specs=[
            pl.BlockSpec((RB4, HID), lambda i: (i, 0)),
            pl.BlockSpec((RB4, HID), lambda i: (i, 0)),
            pl.BlockSpec((RB4, HID), lambda i: (i, 0)),
            pl.BlockSpec((HID, HID), lambda i: (0, 0)),
            pl.BlockSpec((1, HID), lambda i: (0, 0)),
            pl.BlockSpec((HID, HID), lambda i: (0, 0)),
            pl.BlockSpec((1, HID), lambda i: (0, 0)),
            pl.BlockSpec((1, HID), lambda i: (0, 0)),
        ],
        out_specs=[
            pl.BlockSpec((RB4, HID), lambda i: (i, 0)),
            pl.BlockSpec((8, HID), lambda i: (0, 0)),
        ],
        out_shape=[
            jax.ShapeDtypeStruct((N, HID), jnp.float32),
            jax.ShapeDtypeStruct((8, HID), jnp.float32),
        ],
        compiler_params=pltpu.CompilerParams(
            dimension_semantics=("arbitrary",)),
    )(aadj, fs, fmeta, wtopo, btopo, wsem, bsem, qsem)


def _out_body(fmeta_ref, ftopo_ref, wsum_ref, wlin_ref, blin_ref,
              logits_ref, fout_ref):
    wm = wsum_ref[0, 0] * (1.0 / N)
    wt = wsum_ref[1, 0] * (1.0 / N)
    m = jnp.maximum(wm, wt)
    e0 = jnp.exp(wm - m)
    e1 = jnp.exp(wt - m)
    b0 = e0 / (e0 + e1)
    b1 = e1 / (e0 + e1)
    fo = b0 * fmeta_ref[...] + b1 * ftopo_ref[...]
    fout_ref[...] = fo
    logits_ref[...] = fo @ wlin_ref[...] + blin_ref[...]


def _final(fmeta, ftopo, wsum, wlin, blin):
    return pl.pallas_call(
        _out_body,
        grid=(N // RB5,),
        in_specs=[
            pl.BlockSpec((RB5, HID), lambda i: (i, 0)),
            pl.BlockSpec((RB5, HID), lambda i: (i, 0)),
            pl.BlockSpec((8, HID), lambda i: (0, 0)),
            pl.BlockSpec((HID, NUMCLASS), lambda i: (0, 0)),
            pl.BlockSpec((1, NUMCLASS), lambda i: (0, 0)),
        ],
        out_specs=[
            pl.BlockSpec((RB5, NUMCLASS), lambda i: (i, 0)),
            pl.BlockSpec((RB5, HID), lambda i: (i, 0)),
        ],
        out_shape=[
            jax.ShapeDtypeStruct((N, NUMCLASS), jnp.float32),
            jax.ShapeDtypeStruct((N, HID), jnp.float32),
        ],
    )(fmeta, ftopo, wsum, wlin, blin)


def kernel(features, adjM, ADJ, feature_attr, W_trans, b_trans, W_topo, b_topo,
           W_meta, b_meta, W_sem, b_sem, q_sem, W_lin, b_lin,
           feat_similar_neighbors):
    bt = b_trans.reshape(1, HID)
    bm0 = b_meta[0].reshape(1, HID)
    bm1 = b_meta[1].reshape(1, HID)
    btopo = b_topo.reshape(1, HID)
    bsem = b_sem.reshape(1, HID)
    qsem = q_sem.reshape(1, HID)
    blin = b_lin.reshape(1, NUMCLASS)

    feat, feat_pk, aw = _precompute(features, feature_attr, W_trans, bt)

    idx = feat_similar_neighbors.astype(jnp.int32).reshape(-1)
    idx_rows = jnp.pad(idx, (0, (NPAD - N) * TOPO)).reshape(NW * NR, 128)
    aw_pad = jnp.pad(aw, ((0, NPAD - N), (0, 0))).reshape(-1)
    fs = _sc_gather(feat_pk, idx_rows, aw_pad).reshape(NPAD, HID)

    aadj, fmeta = _big(adjM, ADJ, feat, W_meta[0], W_meta[1], bm0, bm1)
    ftopo, wsum = _topo(aadj, fs, fmeta, W_topo, btopo, W_sem, bsem, qsem)
    logits, fout = _final(fmeta, ftopo, wsum, W_lin, blin)
    return (logits, fout)


# speedup vs baseline: 2.5925x; 1.0039x over previous
import functools

import jax
import jax.numpy as jnp
from jax import lax
from jax.experimental import pallas as pl
from jax.experimental.pallas import tpu as pltpu
from jax.experimental.pallas import tpu_sc as plsc

N = 10000
INFEAT = 256
HID = 128
TOPO = 32
NUMCLASS = 64

NW = 32
NPW = 320
NPAD = NW * NPW
CH = 4

RB1 = 2000
RB = 200
RB4 = 2000
RB5 = 2000


def _pre_body(f_ref, fa_ref, wt_ref, bt_ref, feat_ref, pk_ref, aw_ref):
    f = f_ref[...] @ wt_ref[...] + bt_ref[...]
    feat_ref[...] = f
    u = jax.lax.bitcast_convert_type(f.astype(jnp.bfloat16),
                                     jnp.uint16).astype(jnp.int32)
    lo = jnp.concatenate([u[:, g * 32:g * 32 + 16] for g in range(HID // 32)],
                         axis=1)
    hi = jnp.concatenate([u[:, g * 32 + 16:g * 32 + 32]
                          for g in range(HID // 32)], axis=1)
    pk_ref[...] = lo | (hi << 16)
    a = fa_ref[...]
    e = jnp.exp(a - jnp.max(a, axis=1, keepdims=True))
    aw_ref[...] = e / jnp.sum(e, axis=1, keepdims=True)


def _precompute(features, fa, W_trans, bt):
    gi = N // RB1
    return pl.pallas_call(
        _pre_body,
        grid=(gi,),
        in_specs=[
            pl.BlockSpec((RB1, INFEAT), lambda i: (i, 0)),
            pl.BlockSpec((RB1, TOPO), lambda i: (i, 0)),
            pl.BlockSpec((INFEAT, HID), lambda i: (0, 0)),
            pl.BlockSpec((1, HID), lambda i: (0, 0)),
        ],
        out_specs=[
            pl.BlockSpec((RB1, HID), lambda i: (i, 0)),
            pl.BlockSpec((RB1, HID // 2), lambda i: (i, 0)),
            pl.BlockSpec((RB1, TOPO), lambda i: (i, 0)),
        ],
        out_shape=[
            jax.ShapeDtypeStruct((N, HID), jnp.float32),
            jax.ShapeDtypeStruct((N, HID // 2), jnp.int32),
            jax.ShapeDtypeStruct((NPAD, TOPO), jnp.float32),
        ],
    )(features, fa, W_trans, bt)


NR = NPW * TOPO // 128


def _sc_gather(xt, idx_rows, w_pad):
    info = plsc.get_sparse_core_info()
    nc = info.num_cores
    mesh = plsc.VectorSubcoreMesh(core_axis_name="c", subcore_axis_name="s")

    NBUF = 4

    @functools.partial(
        pl.kernel, mesh=mesh,
        compiler_params=pltpu.CompilerParams(use_tc_tiling_on_sc=False),
        out_type=jax.ShapeDtypeStruct((NPAD * HID,), jnp.float32),
        scratch_types=(
            [pltpu.VMEM((128,), jnp.int32) for _ in range(NBUF)]
            + [pltpu.VMEM((CH * TOPO, HID // 2), jnp.int32) for _ in range(NBUF)]
            + [pltpu.VMEM((NPW * TOPO,), jnp.float32),
               pltpu.VMEM((NPW * HID,), jnp.float32),
               pltpu.VMEM_SHARED((N, HID // 2), jnp.int32)]
            + [pltpu.SemaphoreType.DMA for _ in range(2 * NBUF)]
        ),
    )
    def k(xt_hbm, idx_hbm, w_hbm, out_hbm, *sc):
        idxbs = sc[0:NBUF]
        rowbs = sc[NBUF:2 * NBUF]
        w_v, out_v, tbl = sc[2 * NBUF], sc[2 * NBUF + 1], sc[2 * NBUF + 2]
        sis = sc[2 * NBUF + 3:2 * NBUF + 3 + NBUF]
        srs = sc[2 * NBUF + 3 + NBUF:2 * NBUF + 3 + 2 * NBUF]
        wid = lax.axis_index("s") * nc + lax.axis_index("c")
        base = wid * NPW
        row0 = wid * NR

        @pl.when(lax.axis_index("s") == 0)
        def _():
            pltpu.sync_copy(xt_hbm, tbl)

        pltpu.sync_copy(w_hbm.at[pl.ds(base * TOPO, NPW * TOPO)], w_v)
        plsc.subcore_barrier()
        for b in range(NBUF):
            pltpu.sync_copy(idx_hbm.at[row0 + b], idxbs[b])
            pltpu.make_async_copy(tbl.at[idxbs[b]], rowbs[b], srs[b]).start()

        def ring(i, carry):
            c0 = i * NBUF
            for b in range(NBUF):
                idxb, rows, si, sr = idxbs[b], rowbs[b], sis[b], srs[b]
                c = c0 + b
                pltpu.make_async_copy(tbl.at[idxb], rows, sr).wait()

                @pl.when(c + NBUF < NR)
                def _():
                    pltpu.make_async_copy(idx_hbm.at[row0 + c + NBUF], idxb,
                                          si).start()

                def node_body(n, cr):
                    node = c * CH + n
                    accs = [jnp.zeros((16,), jnp.float32) for _ in range(8)]
                    for g in range(TOPO // 16):
                        wv = w_v[pl.ds(node * TOPO + g * 16, 16)]
                        for j in range(16):
                            wgt = wv[j]
                            r = n * TOPO + g * 16 + j
                            for g2 in range(HID // 32):
                                pv32 = rows[r, pl.ds(g2 * 16, 16)]
                                pa = jax.lax.bitcast_convert_type(
                                    pv32 << 16, jnp.float32)
                                pb = jax.lax.bitcast_convert_type(
                                    pv32 & jnp.int32(-65536), jnp.float32)
                                accs[2 * g2] = accs[2 * g2] + wgt * pa
                                accs[2 * g2 + 1] = accs[2 * g2 + 1] + wgt * pb
                    for kk in range(8):
                        out_v[pl.ds(node * HID + kk * 16, 16)] = accs[kk]
                    return cr

                lax.fori_loop(0, CH, node_body, 0)

                @pl.when(c + NBUF < NR)
                def _():
                    pltpu.make_async_copy(idx_hbm.at[row0 + c + NBUF], idxb,
                                          si).wait()
                    pltpu.make_async_copy(tbl.at[idxb], rows, sr).start()
            return carry

        lax.fori_loop(0, NR // NBUF, ring, 0)
        pltpu.sync_copy(out_v, out_hbm.at[pl.ds(base * HID, NPW * HID)])

    return k(xt, idx_rows, w_pad)


def _big_body(adj_ref, a0_ref, a1_ref, feat_ref, wm0_ref, wm1_ref,
              bm0_ref, bm1_ref, aadj_ref, fmeta_ref):
    aadj_ref[...] = adj_ref[...] @ feat_ref[...]
    agg0 = a0_ref[0] @ feat_ref[...]
    agg1 = a1_ref[0] @ feat_ref[...]
    m0 = agg0 @ wm0_ref[...]
    m1 = agg1 @ wm1_ref[...]
    fmeta_ref[...] = 0.5 * (jnp.tanh(m0 + bm0_ref[...]) +
                            jnp.tanh(m1 + bm1_ref[...]))


def _big(adjM, ADJ, feat, wm0, wm1, bm0, bm1):
    return pl.pallas_call(
        _big_body,
        grid=(N // RB,),
        in_specs=[
            pl.BlockSpec((RB, N), lambda i: (i, 0)),
            pl.BlockSpec((1, RB, N), lambda i: (0, i, 0)),
            pl.BlockSpec((1, RB, N), lambda i: (1, i, 0)),
            pl.BlockSpec((N, HID), lambda i: (0, 0)),
            pl.BlockSpec((HID, HID), lambda i: (0, 0)),
            pl.BlockSpec((HID, HID), lambda i: (0, 0)),
            pl.BlockSpec((1, HID), lambda i: (0, 0)),
            pl.BlockSpec((1, HID), lambda i: (0, 0)),
        ],
        out_specs=[
            pl.BlockSpec((RB, HID), lambda i: (i, 0)),
            pl.BlockSpec((RB, HID), lambda i: (i, 0)),
        ],
        out_shape=[
            jax.ShapeDtypeStruct((N, HID), jnp.float32),
            jax.ShapeDtypeStruct((N, HID), jnp.float32),
        ],
        compiler_params=pltpu.CompilerParams(
            dimension_semantics=("parallel",),
            vmem_limit_bytes=120 * 1024 * 1024),
    )(adjM, ADJ, ADJ, feat, wm0, wm1, bm0, bm1)


def _topo_body(aadj_ref, fs_ref, fmeta_ref, wtopo_ref, btopo_ref, wsem_ref,
               bsem_ref, qsem_ref, ftopo_ref, wsum_ref):
    i = pl.program_id(0)

    @pl.when(i == 0)
    def _():
        wsum_ref[...] = jnp.zeros_like(wsum_ref)

    ftopo = jnp.tanh((aadj_ref[...] + fs_ref[...]) @ wtopo_ref[...]
                     + btopo_ref[...])
    ftopo_ref[...] = ftopo
    sm = jnp.sum(jnp.tanh(fmeta_ref[...] @ wsem_ref[...] + bsem_ref[...])
                 * qsem_ref[...])
    st = jnp.sum(jnp.tanh(ftopo @ wsem_ref[...] + bsem_ref[...])
                 * qsem_ref[...])
    upd = jnp.concatenate(
        [jnp.full((1, HID), sm, jnp.float32),
         jnp.full((1, HID), st, jnp.float32),
         jnp.zeros((6, HID), jnp.float32)], axis=0)
    wsum_ref[...] += upd


def _topo(aadj, fs, fmeta, wtopo, btopo, wsem, bsem, qsem):
    return pl.pallas_call(
        _topo_body,
        grid=(N // RB4,),
        in_specs=[
            pl.BlockSpec((RB4, HID), lambda i: (i, 0)),
            pl.BlockSpec((RB4, HID), lambda i: (i, 0)),
            pl.BlockSpec((RB4, HID), lambda i: (i, 0)),
            pl.BlockSpec((HID, HID), lambda i: (0, 0)),
            pl.BlockSpec((1, HID), lambda i: (0, 0)),
            pl.BlockSpec((HID, HID), lambda i: (0, 0)),
            pl.BlockSpec((1, HID), lambda i: (0, 0)),
            pl.BlockSpec((1, HID), lambda i: (0, 0)),
        ],
        out_specs=[
            pl.BlockSpec((RB4, HID), lambda i: (i, 0)),
            pl.BlockSpec((8, HID), lambda i: (0, 0)),
        ],
        out_shape=[
            jax.ShapeDtypeStruct((N, HID), jnp.float32),
            jax.ShapeDtypeStruct((8, HID), jnp.float32),
        ],
        compiler_params=pltpu.CompilerParams(
            dimension_semantics=("arbitrary",)),
    )(aadj, fs, fmeta, wtopo, btopo, wsem, bsem, qsem)


def _out_body(fmeta_ref, ftopo_ref, wsum_ref, wlin_ref, blin_ref,
              logits_ref, fout_ref):
    wm = wsum_ref[0, 0] * (1.0 / N)
    wt = wsum_ref[1, 0] * (1.0 / N)
    m = jnp.maximum(wm, wt)
    e0 = jnp.exp(wm - m)
    e1 = jnp.exp(wt - m)
    b0 = e0 / (e0 + e1)
    b1 = e1 / (e0 + e1)
    fo = b0 * fmeta_ref[...] + b1 * ftopo_ref[...]
    fout_ref[...] = fo
    logits_ref[...] = fo @ wlin_ref[...] + blin_ref[...]


def _final(fmeta, ftopo, wsum, wlin, blin):
    return pl.pallas_call(
        _out_body,
        grid=(N // RB5,),
        in_specs=[
            pl.BlockSpec((RB5, HID), lambda i: (i, 0)),
            pl.BlockSpec((RB5, HID), lambda i: (i, 0)),
            pl.BlockSpec((8, HID), lambda i: (0, 0)),
            pl.BlockSpec((HID, NUMCLASS), lambda i: (0, 0)),
            pl.BlockSpec((1, NUMCLASS), lambda i: (0, 0)),
        ],
        out_specs=[
            pl.BlockSpec((RB5, NUMCLASS), lambda i: (i, 0)),
            pl.BlockSpec((RB5, HID), lambda i: (i, 0)),
        ],
        out_shape=[
            jax.ShapeDtypeStruct((N, NUMCLASS), jnp.float32),
            jax.ShapeDtypeStruct((N, HID), jnp.float32),
        ],
    )(fmeta, ftopo, wsum, wlin, blin)


def kernel(features, adjM, ADJ, feature_attr, W_trans, b_trans, W_topo, b_topo,
           W_meta, b_meta, W_sem, b_sem, q_sem, W_lin, b_lin,
           feat_similar_neighbors):
    bt = b_trans.reshape(1, HID)
    bm0 = b_meta[0].reshape(1, HID)
    bm1 = b_meta[1].reshape(1, HID)
    btopo = b_topo.reshape(1, HID)
    bsem = b_sem.reshape(1, HID)
    qsem = q_sem.reshape(1, HID)
    blin = b_lin.reshape(1, NUMCLASS)

    feat, feat_pk, aw = _precompute(features, feature_attr, W_trans, bt)

    idx = feat_similar_neighbors.astype(jnp.int32).reshape(-1)
    idx_rows = jnp.pad(idx, (0, (NPAD - N) * TOPO)).reshape(NW * NR, 128)
    fs = _sc_gather(feat_pk, idx_rows, aw.reshape(-1)).reshape(NPAD, HID)

    aadj, fmeta = _big(adjM, ADJ, feat, W_meta[0], W_meta[1], bm0, bm1)
    ftopo, wsum = _topo(aadj, fs, fmeta, W_topo, btopo, W_sem, bsem, qsem)
    logits, fout = _final(fmeta, ftopo, wsum, W_lin, blin)
    return (logits, fout)
